# Initial kernel scaffold; baseline (speedup 1.0000x reference)
import jax
import jax.numpy as jnp
from jax.experimental import pallas as pl


def kernel(regressions, anchors, refined_achors, annotations):
    raise NotImplementedError("write your pallas kernel here")


# speedup vs baseline: 21.2489x; 21.2489x over previous
import jax
import jax.numpy as jnp
from jax.experimental import pallas as pl
from jax.experimental.pallas import tpu as pltpu

MD_THRES = 0.5
BETA = 1.0 / 9.0
PI180 = 3.14159265358979323846 / 180.0

LANES = 128
SUBL = 8
TILE = LANES * SUBL
NB = 5
NPAD = TILE * NB
ROWS_PER_BATCH = NPAD // LANES


def _batcher_net(n):
    pairs = []

    def merge(lo, cnt, r):
        step = r * 2
        if step < cnt:
            merge(lo, cnt, step)
            merge(lo + r, cnt, step)
            for i in range(lo + r, lo + cnt - r, step):
                pairs.append((i, i + r))
        else:
            pairs.append((lo, lo + r))

    def sort(lo, cnt):
        if cnt > 1:
            m = cnt // 2
            sort(lo, m)
            sort(lo + m, m)
            merge(lo, cnt, 1)

    p2 = 1 << (n - 1).bit_length()
    sort(0, p2)
    return [(a, b) for (a, b) in pairs if a < n and b < n]


NET24 = _batcher_net(24)


def _rmax2(x):
    return jnp.max(jnp.max(x, axis=1, keepdims=True), axis=0, keepdims=True)


def _rmin2(x):
    return jnp.min(jnp.min(x, axis=1, keepdims=True), axis=0, keepdims=True)


def _rsum2(x):
    return jnp.sum(jnp.sum(x, axis=1, keepdims=True), axis=0, keepdims=True)


def _pairs_kernel(acx_ref, acy_ref, aw_ref, ah_ref, at_ref,
                  r0_ref, r1_ref, r2_ref, r3_ref, r4_ref,
                  gcx_ref, gcy_ref, gw_ref, gh_ref, gtan_ref,
                  sgx0_ref, sgy0_ref, sgx1_ref, sgy1_ref, gasq_ref,
                  cbx0_ref, cbx1_ref, cbx2_ref, cbx3_ref,
                  cby0_ref, cby1_ref, cby2_ref, cby3_ref,
                  gab_ref,
                  l5_ref, posb_ref, colmax_ref, colarg_ref):
    g = pl.program_id(0)
    j = g // NB
    lb = g % NB

    f32 = jnp.float32
    acx = acx_ref[:, :]
    acy = acy_ref[:, :]
    aw = aw_ref[:, :]
    ah = ah_ref[:, :]
    at = at_ref[:, :]

    rowi = jax.lax.broadcasted_iota(jnp.int32, (SUBL, LANES), 0)
    lane = jax.lax.broadcasted_iota(jnp.int32, (SUBL, LANES), 1)
    lidx = lb * TILE + rowi * LANES + lane
    valid = lidx < 5000

    s_a = jnp.maximum(aw, ah)
    ax0 = acx - s_a / 2
    ay0 = acy - s_a / 2
    ax1 = acx + s_a / 2
    ay1 = acy + s_a / 2
    area_asq = (ax1 - ax0) * (ay1 - ay0)
    area_a = aw * ah

    aa = at * PI180
    ca = jnp.cos(aa)
    sa = jnp.sin(aa)
    tan_a = jnp.tan(aa)
    hw = aw / 2
    hh = ah / 2
    DXS = (-1.0, 1.0, 1.0, -1.0)
    DYS = (-1.0, -1.0, 1.0, 1.0)
    AX = [acx + (dx * hw) * ca - (dy * hh) * sa for dx, dy in zip(DXS, DYS)]
    AY = [acy + (dx * hw) * sa + (dy * hh) * ca for dx, dy in zip(DXS, DYS)]
    EAX = [AX[(k + 1) % 4] - AX[k] for k in range(4)]
    EAY = [AY[(k + 1) % 4] - AY[k] for k in range(4)]

    @pl.when(g == 0)
    def _init():
        colmax_ref[:, :] = jnp.full((2, 32), -1.0, f32)
        colarg_ref[:, :] = jnp.zeros((2, 32), jnp.int32)

    cmv0 = colmax_ref[pl.ds(j, 1), :]
    cav0 = colarg_ref[pl.ds(j, 1), :]

    iota32 = jax.lax.broadcasted_iota(jnp.int32, (1, 32), 1)

    def gt_body(m, carry):
        rmax, rarg, cmv, cav = carry
        ltx = jnp.maximum(ax0, sgx0_ref[j, m])
        lty = jnp.maximum(ay0, sgy0_ref[j, m])
        rbx = jnp.minimum(ax1, sgx1_ref[j, m])
        rby = jnp.minimum(ay1, sgy1_ref[j, m])
        iw = jnp.clip(rbx - ltx, 0.0, None)
        ih = jnp.clip(rby - lty, 0.0, None)
        inter_sq = iw * ih
        union_sq = area_asq + gasq_ref[j, m] - inter_sq
        bf = inter_sq / jnp.maximum(union_sq, 1e-9)

        BX = (cbx0_ref[j, m], cbx1_ref[j, m], cbx2_ref[j, m], cbx3_ref[j, m])
        BY = (cby0_ref[j, m], cby1_ref[j, m], cby2_ref[j, m], cby3_ref[j, m])
        EBX = [BX[(k + 1) % 4] - BX[k] for k in range(4)]
        EBY = [BY[(k + 1) % 4] - BY[k] for k in range(4)]

        pts_x, pts_y, mf = [], [], []
        for p in range(4):
            ok = None
            for k in range(4):
                cr = EBX[k] * (AY[p] - BY[k]) - EBY[k] * (AX[p] - BX[k])
                c = cr >= -1e-9
                ok = c if ok is None else (ok & c)
            pts_x.append(AX[p])
            pts_y.append(AY[p])
            mf.append(ok)
        for q in range(4):
            ok = None
            for k in range(4):
                cr = EAX[k] * (jnp.float32(BY[q]) - AY[k]) - EAY[k] * (jnp.float32(BX[q]) - AX[k])
                c = cr >= -1e-9
                ok = c if ok is None else (ok & c)
            pts_x.append(jnp.full((SUBL, LANES), BX[q], f32))
            pts_y.append(jnp.full((SUBL, LANES), BY[q], f32))
            mf.append(ok)
        for p in range(4):
            for q in range(4):
                rx, ry = EAX[p], EAY[p]
                sx, sy = EBX[q], EBY[q]
                qpx = BX[q] - AX[p]
                qpy = BY[q] - AY[p]
                denom = rx * sy - ry * sx
                okd = jnp.abs(denom) > 1e-12
                den = jnp.where(okd, denom, 1.0)
                t = (qpx * sy - qpy * sx) / den
                u = (qpx * ry - qpy * rx) / den
                vv = okd & (t >= 0.0) & (t <= 1.0) & (u >= 0.0) & (u <= 1.0)
                pts_x.append(AX[p] + t * rx)
                pts_y.append(AY[p] + t * ry)
                mf.append(vv)

        mflt = [jnp.where(mm, 1.0, 0.0) for mm in mf]
        cnt = mflt[0]
        for k in range(1, 24):
            cnt = cnt + mflt[k]
        cntc = jnp.maximum(cnt, 1.0)
        ctrx = pts_x[0] * mflt[0]
        ctry = pts_y[0] * mflt[0]
        for k in range(1, 24):
            ctrx = ctrx + pts_x[k] * mflt[k]
            ctry = ctry + pts_y[k] * mflt[k]
        ctrx = ctrx / cntc
        ctry = ctry / cntc

        K = []
        for k in range(24):
            dx = pts_x[k] - ctrx
            dy = pts_y[k] - ctry
            sden = jnp.abs(dx) + jnp.abs(dy)
            r = dx / jnp.where(sden == 0.0, 1.0, sden)
            key = jnp.where(dy >= 0.0, 1.0 - r, r - 1.0)
            K.append(jnp.where(mf[k], key, 1e9))
        X = list(pts_x)
        Y = list(pts_y)
        for a, b in NET24:
            sw = K[a] > K[b]
            ka = jnp.where(sw, K[b], K[a])
            kb = jnp.where(sw, K[a], K[b])
            xa = jnp.where(sw, X[b], X[a])
            xb = jnp.where(sw, X[a], X[b])
            ya = jnp.where(sw, Y[b], Y[a])
            yb = jnp.where(sw, Y[a], Y[b])
            K[a], K[b], X[a], X[b], Y[a], Y[b] = ka, kb, xa, xb, ya, yb
        PX = [jnp.where(cnt > k, X[k], X[0]) for k in range(24)]
        PY = [jnp.where(cnt > k, Y[k], Y[0]) for k in range(24)]
        crs = PX[23] * PY[0] - PY[23] * PX[0]
        for k in range(23):
            crs = crs + (PX[k] * PY[k + 1] - PY[k] * PX[k + 1])
        area = 0.5 * jnp.abs(crs)
        inter = jnp.where(cnt >= 3.0, area, 0.0)
        iou = inter / jnp.maximum(area_a + gab_ref[j, m] - inter, 1e-9)
        md = jnp.where(bf > 0.1, iou, 0.0)

        upd = md > rmax
        rmax = jnp.where(upd, md, rmax)
        rarg = jnp.where(upd, m, rarg)

        mdc = jnp.where(valid, md, -1.0)
        mx = _rmax2(mdc)
        mxb8 = jnp.broadcast_to(mx, (SUBL, LANES))
        cand = _rmin2(jnp.where(mdc == mxb8, lidx, jnp.int32(2 ** 30)))
        mxb = jnp.broadcast_to(mx, (1, 32))
        cnb = jnp.broadcast_to(cand, (1, 32))
        better = (iota32 == m) & (mxb > cmv)
        cmv = jnp.where(better, mxb, cmv)
        cav = jnp.where(better, cnb, cav)
        return rmax, rarg, cmv, cav

    rmax0 = jnp.full((SUBL, LANES), -1.0, f32)
    rarg0 = jnp.zeros((SUBL, LANES), jnp.int32)
    rmax, rarg, cmv, cav = jax.lax.fori_loop(
        0, 32, gt_body, (rmax0, rarg0, cmv0, cav0))

    colmax_ref[pl.ds(j, 1), :] = cmv
    colarg_ref[pl.ds(j, 1), :] = cav

    g0x = jnp.full((SUBL, LANES), gcx_ref[j, 0], f32)
    g0y = jnp.full((SUBL, LANES), gcy_ref[j, 0], f32)
    g0w = jnp.full((SUBL, LANES), gw_ref[j, 0], f32)
    g0h = jnp.full((SUBL, LANES), gh_ref[j, 0], f32)
    g0t = jnp.full((SUBL, LANES), gtan_ref[j, 0], f32)
    for m in range(1, 32):
        sel = rarg == m
        g0x = jnp.where(sel, gcx_ref[j, m], g0x)
        g0y = jnp.where(sel, gcy_ref[j, m], g0y)
        g0w = jnp.where(sel, gw_ref[j, m], g0w)
        g0h = jnp.where(sel, gh_ref[j, m], g0h)
        g0t = jnp.where(sel, gtan_ref[j, m], g0t)

    d0 = 10.0 * (g0x - acx) / aw
    d1 = 10.0 * (g0y - acy) / ah
    d2 = 5.0 * jnp.log(g0w / aw)
    d3 = 5.0 * jnp.log(g0h / ah)
    d4 = 15.0 * (g0t - tan_a)
    l5 = jnp.zeros((SUBL, LANES), f32)
    for t_, r_ in ((d0, r0_ref), (d1, r1_ref), (d2, r2_ref),
                   (d3, r3_ref), (d4, r4_ref)):
        diff = jnp.abs(r_[:, :] - t_)
        l5 = l5 + jnp.where(diff < BETA, 0.5 * diff * diff / BETA,
                            diff - 0.5 * BETA)

    l5_ref[:, :] = jnp.where(valid, l5, 0.0)
    posb_ref[:, :] = jnp.where(valid & (rmax >= MD_THRES), 1.0, 0.0)


def _loss_kernel(l5_ref, posb_ref, colmax_ref, colarg_ref, out_ref):
    R = ROWS_PER_BATCH
    rowi = jax.lax.broadcasted_iota(jnp.int32, (R, LANES), 0)
    lane = jax.lax.broadcasted_iota(jnp.int32, (R, LANES), 1)
    idx = rowi * LANES + lane
    total = jnp.zeros((1, 1), jnp.float32)
    for j in range(2):
        l5 = l5_ref[pl.ds(j * R, R), :]
        pos = posb_ref[pl.ds(j * R, R), :]
        for m in range(32):
            am = colarg_ref[j, m]
            fm = jnp.where(colmax_ref[j, m] < MD_THRES, 1.0, 0.0)
            pos = jnp.maximum(pos, jnp.where(idx == am, fm, 0.0))
        S = _rsum2(pos * l5)
        num = jnp.maximum(_rsum2(pos), 1.0)
        total = total + S / (num * 5.0)
    out_ref[:, :] = total * 0.5


def kernel(regressions, anchors, refined_achors, annotations):
    f32 = jnp.float32
    B, N, _ = anchors.shape

    def acomp(x, c, pad):
        v = x[:, :, c]
        v = jnp.pad(v, ((0, 0), (0, NPAD - N)), constant_values=pad)
        return v.reshape(B * ROWS_PER_BATCH, LANES)

    a_in = [acomp(anchors, c, 1.0) for c in range(5)]
    r_in = [acomp(regressions, c, 0.0) for c in range(5)]

    gcx = annotations[:, :, 0]
    gcy = annotations[:, :, 1]
    gw = annotations[:, :, 2]
    gh = annotations[:, :, 3]
    gt = annotations[:, :, 4]
    s_g = jnp.maximum(gw, gh)
    sgx0 = gcx - s_g / 2
    sgy0 = gcy - s_g / 2
    sgx1 = gcx + s_g / 2
    sgy1 = gcy + s_g / 2
    gasq = (sgx1 - sgx0) * (sgy1 - sgy0)
    ga = gt * (jnp.pi / 180.0)
    cg, sg = jnp.cos(ga), jnp.sin(ga)
    gtan = jnp.tan(ga)
    cbx = []
    cby = []
    for dx, dy in ((-0.5, -0.5), (0.5, -0.5), (0.5, 0.5), (-0.5, 0.5)):
        cbx.append(gcx + (dx * gw) * cg - (dy * gh) * sg)
        cby.append(gcy + (dx * gw) * sg + (dy * gh) * cg)
    gab = gw * gh

    smem_arrays = [gcx, gcy, gw, gh, gtan, sgx0, sgy0, sgx1, sgy1, gasq,
                   cbx[0], cbx[1], cbx[2], cbx[3],
                   cby[0], cby[1], cby[2], cby[3], gab]

    vspec = pl.BlockSpec((SUBL, LANES), lambda g: (g, 0))
    sspec = pl.BlockSpec(memory_space=pltpu.SMEM)
    fullspec = pl.BlockSpec((2, 32), lambda g: (0, 0))

    l5, posb, colmax, colarg = pl.pallas_call(
        _pairs_kernel,
        grid=(B * NB,),
        in_specs=[vspec] * 10 + [sspec] * 19,
        out_specs=[
            vspec, vspec, fullspec, fullspec,
        ],
        out_shape=[
            jax.ShapeDtypeStruct((B * ROWS_PER_BATCH, LANES), f32),
            jax.ShapeDtypeStruct((B * ROWS_PER_BATCH, LANES), f32),
            jax.ShapeDtypeStruct((2, 32), f32),
            jax.ShapeDtypeStruct((2, 32), jnp.int32),
        ],
    )(*a_in, *r_in, *smem_arrays)

    loss = pl.pallas_call(
        _loss_kernel,
        in_specs=[
            pl.BlockSpec(memory_space=pltpu.VMEM),
            pl.BlockSpec(memory_space=pltpu.VMEM),
            sspec, sspec,
        ],
        out_specs=pl.BlockSpec(memory_space=pltpu.VMEM),
        out_shape=jax.ShapeDtypeStruct((1, 1), f32),
    )(l5, posb, colmax, colarg)
    return loss.reshape(1)


# speedup vs baseline: 26.9597x; 1.2688x over previous
import jax
import jax.numpy as jnp
from jax.experimental import pallas as pl
from jax.experimental.pallas import tpu as pltpu

MD_THRES = 0.5
BETA = 1.0 / 9.0
PI180 = 3.14159265358979323846 / 180.0

LANES = 128
SUBL = 8
TILE = LANES * SUBL
NB = 5
NPAD = TILE * NB
ROWS_PER_BATCH = NPAD // LANES


def _batcher_net(n):
    pairs = []

    def merge(lo, cnt, r):
        step = r * 2
        if step < cnt:
            merge(lo, cnt, step)
            merge(lo + r, cnt, step)
            for i in range(lo + r, lo + cnt - r, step):
                pairs.append((i, i + r))
        else:
            pairs.append((lo, lo + r))

    def sort(lo, cnt):
        if cnt > 1:
            m = cnt // 2
            sort(lo, m)
            sort(lo + m, m)
            merge(lo, cnt, 1)

    p2 = 1 << (n - 1).bit_length()
    sort(0, p2)
    return [(a, b) for (a, b) in pairs if a < n and b < n]


NET24 = _batcher_net(24)


def _rmax2(x):
    return jnp.max(jnp.max(x, axis=1, keepdims=True), axis=0, keepdims=True)


def _rmin2(x):
    return jnp.min(jnp.min(x, axis=1, keepdims=True), axis=0, keepdims=True)


def _rsum2(x):
    return jnp.sum(jnp.sum(x, axis=1, keepdims=True), axis=0, keepdims=True)


def _pairs_kernel(acx_ref, acy_ref, aw_ref, ah_ref, at_ref,
                  r0_ref, r1_ref, r2_ref, r3_ref, r4_ref,
                  gcx_ref, gcy_ref, gw_ref, gh_ref, gtan_ref,
                  sgx0_ref, sgy0_ref, sgx1_ref, sgy1_ref, gasq_ref,
                  cbx0_ref, cbx1_ref, cbx2_ref, cbx3_ref,
                  cby0_ref, cby1_ref, cby2_ref, cby3_ref,
                  gab_ref,
                  l5_ref, posb_ref, colmax_ref, colarg_ref):
    g = pl.program_id(0)
    j = g // NB
    lb = g % NB

    f32 = jnp.float32
    acx = acx_ref[:, :]
    acy = acy_ref[:, :]
    aw = aw_ref[:, :]
    ah = ah_ref[:, :]
    at = at_ref[:, :]

    rowi = jax.lax.broadcasted_iota(jnp.int32, (SUBL, LANES), 0)
    lane = jax.lax.broadcasted_iota(jnp.int32, (SUBL, LANES), 1)
    lidx = lb * TILE + rowi * LANES + lane
    valid = lidx < 5000

    s_a = jnp.maximum(aw, ah)
    ax0 = acx - s_a / 2
    ay0 = acy - s_a / 2
    ax1 = acx + s_a / 2
    ay1 = acy + s_a / 2
    area_asq = (ax1 - ax0) * (ay1 - ay0)
    area_a = aw * ah

    aa = at * PI180
    ca = jnp.cos(aa)
    sa = jnp.sin(aa)
    tan_a = jnp.tan(aa)
    hw = aw / 2
    hh = ah / 2
    DXS = (-1.0, 1.0, 1.0, -1.0)
    DYS = (-1.0, -1.0, 1.0, 1.0)
    AX = [acx + (dx * hw) * ca - (dy * hh) * sa for dx, dy in zip(DXS, DYS)]
    AY = [acy + (dx * hw) * sa + (dy * hh) * ca for dx, dy in zip(DXS, DYS)]
    EAX = [AX[(k + 1) % 4] - AX[k] for k in range(4)]
    EAY = [AY[(k + 1) % 4] - AY[k] for k in range(4)]

    @pl.when(g == 0)
    def _init():
        colmax_ref[:, :] = jnp.full((2, 32), -1.0, f32)
        colarg_ref[:, :] = jnp.zeros((2, 32), jnp.int32)

    cmv0 = colmax_ref[pl.ds(j, 1), :]
    cav0 = colarg_ref[pl.ds(j, 1), :]

    iota32 = jax.lax.broadcasted_iota(jnp.int32, (1, 32), 1)

    def compute_md(m):
        ltx = jnp.maximum(ax0, sgx0_ref[j, m])
        lty = jnp.maximum(ay0, sgy0_ref[j, m])
        rbx = jnp.minimum(ax1, sgx1_ref[j, m])
        rby = jnp.minimum(ay1, sgy1_ref[j, m])
        iw = jnp.clip(rbx - ltx, 0.0, None)
        ih = jnp.clip(rby - lty, 0.0, None)
        inter_sq = iw * ih
        union_sq = area_asq + gasq_ref[j, m] - inter_sq
        bf = inter_sq / jnp.maximum(union_sq, 1e-9)

        BX = (cbx0_ref[j, m], cbx1_ref[j, m], cbx2_ref[j, m], cbx3_ref[j, m])
        BY = (cby0_ref[j, m], cby1_ref[j, m], cby2_ref[j, m], cby3_ref[j, m])
        EBX = [BX[(k + 1) % 4] - BX[k] for k in range(4)]
        EBY = [BY[(k + 1) % 4] - BY[k] for k in range(4)]

        pts_x, pts_y, mf = [], [], []
        for p in range(4):
            ok = None
            for k in range(4):
                cr = EBX[k] * (AY[p] - BY[k]) - EBY[k] * (AX[p] - BX[k])
                c = cr >= -1e-9
                ok = c if ok is None else (ok & c)
            pts_x.append(AX[p])
            pts_y.append(AY[p])
            mf.append(ok)
        for q in range(4):
            ok = None
            for k in range(4):
                cr = EAX[k] * (jnp.float32(BY[q]) - AY[k]) - EAY[k] * (jnp.float32(BX[q]) - AX[k])
                c = cr >= -1e-9
                ok = c if ok is None else (ok & c)
            pts_x.append(jnp.full((SUBL, LANES), BX[q], f32))
            pts_y.append(jnp.full((SUBL, LANES), BY[q], f32))
            mf.append(ok)
        for p in range(4):
            for q in range(4):
                rx, ry = EAX[p], EAY[p]
                sx, sy = EBX[q], EBY[q]
                qpx = BX[q] - AX[p]
                qpy = BY[q] - AY[p]
                denom = rx * sy - ry * sx
                okd = jnp.abs(denom) > 1e-12
                den = jnp.where(okd, denom, 1.0)
                t = (qpx * sy - qpy * sx) / den
                u = (qpx * ry - qpy * rx) / den
                vv = okd & (t >= 0.0) & (t <= 1.0) & (u >= 0.0) & (u <= 1.0)
                pts_x.append(AX[p] + t * rx)
                pts_y.append(AY[p] + t * ry)
                mf.append(vv)

        mflt = [jnp.where(mm, 1.0, 0.0) for mm in mf]
        cnt = mflt[0]
        for k in range(1, 24):
            cnt = cnt + mflt[k]
        cntc = jnp.maximum(cnt, 1.0)
        ctrx = pts_x[0] * mflt[0]
        ctry = pts_y[0] * mflt[0]
        for k in range(1, 24):
            ctrx = ctrx + pts_x[k] * mflt[k]
            ctry = ctry + pts_y[k] * mflt[k]
        ctrx = ctrx / cntc
        ctry = ctry / cntc

        K = []
        for k in range(24):
            dx = pts_x[k] - ctrx
            dy = pts_y[k] - ctry
            sden = jnp.abs(dx) + jnp.abs(dy)
            r = dx / jnp.where(sden == 0.0, 1.0, sden)
            key = jnp.where(dy >= 0.0, 1.0 - r, r - 1.0)
            K.append(jnp.where(mf[k], key, 1e9))
        X = list(pts_x)
        Y = list(pts_y)
        for a, b in NET24:
            sw = K[a] > K[b]
            ka = jnp.where(sw, K[b], K[a])
            kb = jnp.where(sw, K[a], K[b])
            xa = jnp.where(sw, X[b], X[a])
            xb = jnp.where(sw, X[a], X[b])
            ya = jnp.where(sw, Y[b], Y[a])
            yb = jnp.where(sw, Y[a], Y[b])
            K[a], K[b], X[a], X[b], Y[a], Y[b] = ka, kb, xa, xb, ya, yb
        PX = [jnp.where(cnt > k, X[k], X[0]) for k in range(24)]
        PY = [jnp.where(cnt > k, Y[k], Y[0]) for k in range(24)]
        crs = PX[23] * PY[0] - PY[23] * PX[0]
        for k in range(23):
            crs = crs + (PX[k] * PY[k + 1] - PY[k] * PX[k + 1])
        area = 0.5 * jnp.abs(crs)
        inter = jnp.where(cnt >= 3.0, area, 0.0)
        iou = inter / jnp.maximum(area_a + gab_ref[j, m] - inter, 1e-9)
        return jnp.where(bf > 0.1, iou, 0.0)

    def gt_body(i, carry):
        rmax, rarg, cmv, cav = carry
        mds = [compute_md(2 * i), compute_md(2 * i + 1)]
        for t in range(2):
            m = 2 * i + t
            md = mds[t]
            upd = md > rmax
            rmax = jnp.where(upd, md, rmax)
            rarg = jnp.where(upd, m, rarg)
            mdc = jnp.where(valid, md, -1.0)
            mx = _rmax2(mdc)
            mxb8 = jnp.broadcast_to(mx, (SUBL, LANES))
            cand = _rmin2(jnp.where(mdc == mxb8, lidx, jnp.int32(2 ** 30)))
            mxb = jnp.broadcast_to(mx, (1, 32))
            cnb = jnp.broadcast_to(cand, (1, 32))
            better = (iota32 == m) & (mxb > cmv)
            cmv = jnp.where(better, mxb, cmv)
            cav = jnp.where(better, cnb, cav)
        return rmax, rarg, cmv, cav

    rmax0 = jnp.full((SUBL, LANES), -1.0, f32)
    rarg0 = jnp.zeros((SUBL, LANES), jnp.int32)
    rmax, rarg, cmv, cav = jax.lax.fori_loop(
        0, 16, gt_body, (rmax0, rarg0, cmv0, cav0))

    colmax_ref[pl.ds(j, 1), :] = cmv
    colarg_ref[pl.ds(j, 1), :] = cav

    g0x = jnp.full((SUBL, LANES), gcx_ref[j, 0], f32)
    g0y = jnp.full((SUBL, LANES), gcy_ref[j, 0], f32)
    g0w = jnp.full((SUBL, LANES), gw_ref[j, 0], f32)
    g0h = jnp.full((SUBL, LANES), gh_ref[j, 0], f32)
    g0t = jnp.full((SUBL, LANES), gtan_ref[j, 0], f32)
    for m in range(1, 32):
        sel = rarg == m
        g0x = jnp.where(sel, gcx_ref[j, m], g0x)
        g0y = jnp.where(sel, gcy_ref[j, m], g0y)
        g0w = jnp.where(sel, gw_ref[j, m], g0w)
        g0h = jnp.where(sel, gh_ref[j, m], g0h)
        g0t = jnp.where(sel, gtan_ref[j, m], g0t)

    d0 = 10.0 * (g0x - acx) / aw
    d1 = 10.0 * (g0y - acy) / ah
    d2 = 5.0 * jnp.log(g0w / aw)
    d3 = 5.0 * jnp.log(g0h / ah)
    d4 = 15.0 * (g0t - tan_a)
    l5 = jnp.zeros((SUBL, LANES), f32)
    for t_, r_ in ((d0, r0_ref), (d1, r1_ref), (d2, r2_ref),
                   (d3, r3_ref), (d4, r4_ref)):
        diff = jnp.abs(r_[:, :] - t_)
        l5 = l5 + jnp.where(diff < BETA, 0.5 * diff * diff / BETA,
                            diff - 0.5 * BETA)

    l5_ref[:, :] = jnp.where(valid, l5, 0.0)
    posb_ref[:, :] = jnp.where(valid & (rmax >= MD_THRES), 1.0, 0.0)


def _loss_kernel(l5_ref, posb_ref, colmax_ref, colarg_ref, out_ref):
    R = ROWS_PER_BATCH
    rowi = jax.lax.broadcasted_iota(jnp.int32, (R, LANES), 0)
    lane = jax.lax.broadcasted_iota(jnp.int32, (R, LANES), 1)
    idx = rowi * LANES + lane
    total = jnp.zeros((1, 1), jnp.float32)
    for j in range(2):
        l5 = l5_ref[pl.ds(j * R, R), :]
        pos = posb_ref[pl.ds(j * R, R), :]
        for m in range(32):
            am = colarg_ref[j, m]
            fm = jnp.where(colmax_ref[j, m] < MD_THRES, 1.0, 0.0)
            pos = jnp.maximum(pos, jnp.where(idx == am, fm, 0.0))
        S = _rsum2(pos * l5)
        num = jnp.maximum(_rsum2(pos), 1.0)
        total = total + S / (num * 5.0)
    out_ref[:, :] = total * 0.5


def kernel(regressions, anchors, refined_achors, annotations):
    f32 = jnp.float32
    B, N, _ = anchors.shape

    def acomp(x, c, pad):
        v = x[:, :, c]
        v = jnp.pad(v, ((0, 0), (0, NPAD - N)), constant_values=pad)
        return v.reshape(B * ROWS_PER_BATCH, LANES)

    a_in = [acomp(anchors, c, 1.0) for c in range(5)]
    r_in = [acomp(regressions, c, 0.0) for c in range(5)]

    gcx = annotations[:, :, 0]
    gcy = annotations[:, :, 1]
    gw = annotations[:, :, 2]
    gh = annotations[:, :, 3]
    gt = annotations[:, :, 4]
    s_g = jnp.maximum(gw, gh)
    sgx0 = gcx - s_g / 2
    sgy0 = gcy - s_g / 2
    sgx1 = gcx + s_g / 2
    sgy1 = gcy + s_g / 2
    gasq = (sgx1 - sgx0) * (sgy1 - sgy0)
    ga = gt * (jnp.pi / 180.0)
    cg, sg = jnp.cos(ga), jnp.sin(ga)
    gtan = jnp.tan(ga)
    cbx = []
    cby = []
    for dx, dy in ((-0.5, -0.5), (0.5, -0.5), (0.5, 0.5), (-0.5, 0.5)):
        cbx.append(gcx + (dx * gw) * cg - (dy * gh) * sg)
        cby.append(gcy + (dx * gw) * sg + (dy * gh) * cg)
    gab = gw * gh

    smem_arrays = [gcx, gcy, gw, gh, gtan, sgx0, sgy0, sgx1, sgy1, gasq,
                   cbx[0], cbx[1], cbx[2], cbx[3],
                   cby[0], cby[1], cby[2], cby[3], gab]

    vspec = pl.BlockSpec((SUBL, LANES), lambda g: (g, 0))
    sspec = pl.BlockSpec(memory_space=pltpu.SMEM)
    fullspec = pl.BlockSpec((2, 32), lambda g: (0, 0))

    l5, posb, colmax, colarg = pl.pallas_call(
        _pairs_kernel,
        grid=(B * NB,),
        in_specs=[vspec] * 10 + [sspec] * 19,
        out_specs=[
            vspec, vspec, fullspec, fullspec,
        ],
        out_shape=[
            jax.ShapeDtypeStruct((B * ROWS_PER_BATCH, LANES), f32),
            jax.ShapeDtypeStruct((B * ROWS_PER_BATCH, LANES), f32),
            jax.ShapeDtypeStruct((2, 32), f32),
            jax.ShapeDtypeStruct((2, 32), jnp.int32),
        ],
    )(*a_in, *r_in, *smem_arrays)

    loss = pl.pallas_call(
        _loss_kernel,
        in_specs=[
            pl.BlockSpec(memory_space=pltpu.VMEM),
            pl.BlockSpec(memory_space=pltpu.VMEM),
            sspec, sspec,
        ],
        out_specs=pl.BlockSpec(memory_space=pltpu.VMEM),
        out_shape=jax.ShapeDtypeStruct((1, 1), f32),
    )(l5, posb, colmax, colarg)
    return loss.reshape(1)


# speedup vs baseline: 31.0577x; 1.1520x over previous
import jax
import jax.numpy as jnp
from jax.experimental import pallas as pl
from jax.experimental.pallas import tpu as pltpu

MD_THRES = 0.5
BETA = 1.0 / 9.0
PI180 = 3.14159265358979323846 / 180.0

LANES = 128
SUBL = 8
TILE = LANES * SUBL
NB = 5
NPAD = TILE * NB
ROWS_PER_BATCH = NPAD // LANES


def _batcher_net(n):
    pairs = []

    def merge(lo, cnt, r):
        step = r * 2
        if step < cnt:
            merge(lo, cnt, step)
            merge(lo + r, cnt, step)
            for i in range(lo + r, lo + cnt - r, step):
                pairs.append((i, i + r))
        else:
            pairs.append((lo, lo + r))

    def sort(lo, cnt):
        if cnt > 1:
            m = cnt // 2
            sort(lo, m)
            sort(lo + m, m)
            merge(lo, cnt, 1)

    p2 = 1 << (n - 1).bit_length()
    sort(0, p2)
    return [(a, b) for (a, b) in pairs if a < n and b < n]


NET24 = _batcher_net(24)


def _rmax2(x):
    return jnp.max(jnp.max(x, axis=1, keepdims=True), axis=0, keepdims=True)


def _rmin2(x):
    return jnp.min(jnp.min(x, axis=1, keepdims=True), axis=0, keepdims=True)


def _rsum2(x):
    return jnp.sum(jnp.sum(x, axis=1, keepdims=True), axis=0, keepdims=True)


def _pairs_kernel(acx_ref, acy_ref, aw_ref, ah_ref, at_ref,
                  r0_ref, r1_ref, r2_ref, r3_ref, r4_ref,
                  gcx_ref, gcy_ref, gw_ref, gh_ref, gtan_ref,
                  sgx0_ref, sgy0_ref, sgx1_ref, sgy1_ref, gasq_ref,
                  cbx0_ref, cbx1_ref, cbx2_ref, cbx3_ref,
                  cby0_ref, cby1_ref, cby2_ref, cby3_ref,
                  gab_ref,
                  l5_ref, posb_ref, colmax_ref, colarg_ref):
    g = pl.program_id(0)
    j = g // NB
    lb = g % NB

    f32 = jnp.float32
    acx = acx_ref[:, :]
    acy = acy_ref[:, :]
    aw = aw_ref[:, :]
    ah = ah_ref[:, :]
    at = at_ref[:, :]

    rowi = jax.lax.broadcasted_iota(jnp.int32, (SUBL, LANES), 0)
    lane = jax.lax.broadcasted_iota(jnp.int32, (SUBL, LANES), 1)
    lidx = lb * TILE + rowi * LANES + lane
    valid = lidx < 5000

    s_a = jnp.maximum(aw, ah)
    ax0 = acx - s_a / 2
    ay0 = acy - s_a / 2
    ax1 = acx + s_a / 2
    ay1 = acy + s_a / 2
    area_asq = (ax1 - ax0) * (ay1 - ay0)
    area_a = aw * ah

    aa = at * PI180
    ca = jnp.cos(aa)
    sa = jnp.sin(aa)
    tan_a = jnp.tan(aa)
    hw = aw / 2
    hh = ah / 2
    DXS = (-1.0, 1.0, 1.0, -1.0)
    DYS = (-1.0, -1.0, 1.0, 1.0)
    AX = [acx + (dx * hw) * ca - (dy * hh) * sa for dx, dy in zip(DXS, DYS)]
    AY = [acy + (dx * hw) * sa + (dy * hh) * ca for dx, dy in zip(DXS, DYS)]
    EAX = [AX[(k + 1) % 4] - AX[k] for k in range(4)]
    EAY = [AY[(k + 1) % 4] - AY[k] for k in range(4)]

    @pl.when(g == 0)
    def _init():
        colmax_ref[:, :] = jnp.full((2, 32), -1.0, f32)
        colarg_ref[:, :] = jnp.zeros((2, 32), jnp.int32)

    cmv0 = colmax_ref[pl.ds(j, 1), :]
    cav0 = colarg_ref[pl.ds(j, 1), :]

    iota32 = jax.lax.broadcasted_iota(jnp.int32, (1, 32), 1)

    def compute_md(m):
        ltx = jnp.maximum(ax0, sgx0_ref[j, m])
        lty = jnp.maximum(ay0, sgy0_ref[j, m])
        rbx = jnp.minimum(ax1, sgx1_ref[j, m])
        rby = jnp.minimum(ay1, sgy1_ref[j, m])
        iw = jnp.clip(rbx - ltx, 0.0, None)
        ih = jnp.clip(rby - lty, 0.0, None)
        inter_sq = iw * ih
        union_sq = area_asq + gasq_ref[j, m] - inter_sq
        bf = inter_sq / jnp.maximum(union_sq, 1e-9)

        BX = (cbx0_ref[j, m], cbx1_ref[j, m], cbx2_ref[j, m], cbx3_ref[j, m])
        BY = (cby0_ref[j, m], cby1_ref[j, m], cby2_ref[j, m], cby3_ref[j, m])
        EBX = [BX[(k + 1) % 4] - BX[k] for k in range(4)]
        EBY = [BY[(k + 1) % 4] - BY[k] for k in range(4)]

        pts_x, pts_y, mf = [], [], []
        for p in range(4):
            ok = None
            for k in range(4):
                cr = EBX[k] * (AY[p] - BY[k]) - EBY[k] * (AX[p] - BX[k])
                c = cr >= -1e-9
                ok = c if ok is None else (ok & c)
            pts_x.append(AX[p])
            pts_y.append(AY[p])
            mf.append(ok)
        for q in range(4):
            ok = None
            for k in range(4):
                cr = EAX[k] * (jnp.float32(BY[q]) - AY[k]) - EAY[k] * (jnp.float32(BX[q]) - AX[k])
                c = cr >= -1e-9
                ok = c if ok is None else (ok & c)
            pts_x.append(jnp.full((SUBL, LANES), BX[q], f32))
            pts_y.append(jnp.full((SUBL, LANES), BY[q], f32))
            mf.append(ok)
        for p in range(4):
            for q in range(4):
                rx, ry = EAX[p], EAY[p]
                sx, sy = EBX[q], EBY[q]
                qpx = BX[q] - AX[p]
                qpy = BY[q] - AY[p]
                denom = rx * sy - ry * sx
                okd = jnp.abs(denom) > 1e-12
                den = jnp.where(okd, denom, 1.0)
                t = (qpx * sy - qpy * sx) / den
                u = (qpx * ry - qpy * rx) / den
                vv = okd & (t >= 0.0) & (t <= 1.0) & (u >= 0.0) & (u <= 1.0)
                pts_x.append(AX[p] + t * rx)
                pts_y.append(AY[p] + t * ry)
                mf.append(vv)

        mflt = [jnp.where(mm, 1.0, 0.0) for mm in mf]
        cnt = mflt[0]
        for k in range(1, 24):
            cnt = cnt + mflt[k]
        cntc = jnp.maximum(cnt, 1.0)
        ctrx = pts_x[0] * mflt[0]
        ctry = pts_y[0] * mflt[0]
        for k in range(1, 24):
            ctrx = ctrx + pts_x[k] * mflt[k]
            ctry = ctry + pts_y[k] * mflt[k]
        ctrx = ctrx / cntc
        ctry = ctry / cntc

        K = []
        for k in range(24):
            dx = pts_x[k] - ctrx
            dy = pts_y[k] - ctry
            sden = jnp.abs(dx) + jnp.abs(dy)
            r = dx / jnp.where(sden == 0.0, 1.0, sden)
            key = jnp.where(dy >= 0.0, 1.0 - r, r - 1.0)
            K.append(jnp.where(mf[k], key, 1e9))
        X = list(pts_x)
        Y = list(pts_y)
        for a, b in NET24:
            sw = K[a] > K[b]
            ka = jnp.where(sw, K[b], K[a])
            kb = jnp.where(sw, K[a], K[b])
            xa = jnp.where(sw, X[b], X[a])
            xb = jnp.where(sw, X[a], X[b])
            ya = jnp.where(sw, Y[b], Y[a])
            yb = jnp.where(sw, Y[a], Y[b])
            K[a], K[b], X[a], X[b], Y[a], Y[b] = ka, kb, xa, xb, ya, yb
        PX = [jnp.where(cnt > k, X[k], X[0]) for k in range(24)]
        PY = [jnp.where(cnt > k, Y[k], Y[0]) for k in range(24)]
        crs = PX[23] * PY[0] - PY[23] * PX[0]
        for k in range(23):
            crs = crs + (PX[k] * PY[k + 1] - PY[k] * PX[k + 1])
        area = 0.5 * jnp.abs(crs)
        inter = jnp.where(cnt >= 3.0, area, 0.0)
        iou = inter / jnp.maximum(area_a + gab_ref[j, m] - inter, 1e-9)
        return jnp.where(bf > 0.1, iou, 0.0)

    def gt_body(i, carry):
        rmax, rarg, cmv, cav = carry
        mds = [compute_md(4 * i + t) for t in range(4)]
        for t in range(4):
            m = 4 * i + t
            md = mds[t]
            upd = md > rmax
            rmax = jnp.where(upd, md, rmax)
            rarg = jnp.where(upd, m, rarg)
            mdc = jnp.where(valid, md, -1.0)
            mx = _rmax2(mdc)
            mxb8 = jnp.broadcast_to(mx, (SUBL, LANES))
            cand = _rmin2(jnp.where(mdc == mxb8, lidx, jnp.int32(2 ** 30)))
            mxb = jnp.broadcast_to(mx, (1, 32))
            cnb = jnp.broadcast_to(cand, (1, 32))
            better = (iota32 == m) & (mxb > cmv)
            cmv = jnp.where(better, mxb, cmv)
            cav = jnp.where(better, cnb, cav)
        return rmax, rarg, cmv, cav

    rmax0 = jnp.full((SUBL, LANES), -1.0, f32)
    rarg0 = jnp.zeros((SUBL, LANES), jnp.int32)
    rmax, rarg, cmv, cav = jax.lax.fori_loop(
        0, 8, gt_body, (rmax0, rarg0, cmv0, cav0))

    colmax_ref[pl.ds(j, 1), :] = cmv
    colarg_ref[pl.ds(j, 1), :] = cav

    g0x = jnp.full((SUBL, LANES), gcx_ref[j, 0], f32)
    g0y = jnp.full((SUBL, LANES), gcy_ref[j, 0], f32)
    g0w = jnp.full((SUBL, LANES), gw_ref[j, 0], f32)
    g0h = jnp.full((SUBL, LANES), gh_ref[j, 0], f32)
    g0t = jnp.full((SUBL, LANES), gtan_ref[j, 0], f32)
    for m in range(1, 32):
        sel = rarg == m
        g0x = jnp.where(sel, gcx_ref[j, m], g0x)
        g0y = jnp.where(sel, gcy_ref[j, m], g0y)
        g0w = jnp.where(sel, gw_ref[j, m], g0w)
        g0h = jnp.where(sel, gh_ref[j, m], g0h)
        g0t = jnp.where(sel, gtan_ref[j, m], g0t)

    d0 = 10.0 * (g0x - acx) / aw
    d1 = 10.0 * (g0y - acy) / ah
    d2 = 5.0 * jnp.log(g0w / aw)
    d3 = 5.0 * jnp.log(g0h / ah)
    d4 = 15.0 * (g0t - tan_a)
    l5 = jnp.zeros((SUBL, LANES), f32)
    for t_, r_ in ((d0, r0_ref), (d1, r1_ref), (d2, r2_ref),
                   (d3, r3_ref), (d4, r4_ref)):
        diff = jnp.abs(r_[:, :] - t_)
        l5 = l5 + jnp.where(diff < BETA, 0.5 * diff * diff / BETA,
                            diff - 0.5 * BETA)

    l5_ref[:, :] = jnp.where(valid, l5, 0.0)
    posb_ref[:, :] = jnp.where(valid & (rmax >= MD_THRES), 1.0, 0.0)


def _loss_kernel(l5_ref, posb_ref, colmax_ref, colarg_ref, out_ref):
    R = ROWS_PER_BATCH
    rowi = jax.lax.broadcasted_iota(jnp.int32, (R, LANES), 0)
    lane = jax.lax.broadcasted_iota(jnp.int32, (R, LANES), 1)
    idx = rowi * LANES + lane
    total = jnp.zeros((1, 1), jnp.float32)
    for j in range(2):
        l5 = l5_ref[pl.ds(j * R, R), :]
        pos = posb_ref[pl.ds(j * R, R), :]
        for m in range(32):
            am = colarg_ref[j, m]
            fm = jnp.where(colmax_ref[j, m] < MD_THRES, 1.0, 0.0)
            pos = jnp.maximum(pos, jnp.where(idx == am, fm, 0.0))
        S = _rsum2(pos * l5)
        num = jnp.maximum(_rsum2(pos), 1.0)
        total = total + S / (num * 5.0)
    out_ref[:, :] = total * 0.5


def kernel(regressions, anchors, refined_achors, annotations):
    f32 = jnp.float32
    B, N, _ = anchors.shape

    def acomp(x, c, pad):
        v = x[:, :, c]
        v = jnp.pad(v, ((0, 0), (0, NPAD - N)), constant_values=pad)
        return v.reshape(B * ROWS_PER_BATCH, LANES)

    a_in = [acomp(anchors, c, 1.0) for c in range(5)]
    r_in = [acomp(regressions, c, 0.0) for c in range(5)]

    gcx = annotations[:, :, 0]
    gcy = annotations[:, :, 1]
    gw = annotations[:, :, 2]
    gh = annotations[:, :, 3]
    gt = annotations[:, :, 4]
    s_g = jnp.maximum(gw, gh)
    sgx0 = gcx - s_g / 2
    sgy0 = gcy - s_g / 2
    sgx1 = gcx + s_g / 2
    sgy1 = gcy + s_g / 2
    gasq = (sgx1 - sgx0) * (sgy1 - sgy0)
    ga = gt * (jnp.pi / 180.0)
    cg, sg = jnp.cos(ga), jnp.sin(ga)
    gtan = jnp.tan(ga)
    cbx = []
    cby = []
    for dx, dy in ((-0.5, -0.5), (0.5, -0.5), (0.5, 0.5), (-0.5, 0.5)):
        cbx.append(gcx + (dx * gw) * cg - (dy * gh) * sg)
        cby.append(gcy + (dx * gw) * sg + (dy * gh) * cg)
    gab = gw * gh

    smem_arrays = [gcx, gcy, gw, gh, gtan, sgx0, sgy0, sgx1, sgy1, gasq,
                   cbx[0], cbx[1], cbx[2], cbx[3],
                   cby[0], cby[1], cby[2], cby[3], gab]

    vspec = pl.BlockSpec((SUBL, LANES), lambda g: (g, 0))
    sspec = pl.BlockSpec(memory_space=pltpu.SMEM)
    fullspec = pl.BlockSpec((2, 32), lambda g: (0, 0))

    l5, posb, colmax, colarg = pl.pallas_call(
        _pairs_kernel,
        grid=(B * NB,),
        in_specs=[vspec] * 10 + [sspec] * 19,
        out_specs=[
            vspec, vspec, fullspec, fullspec,
        ],
        out_shape=[
            jax.ShapeDtypeStruct((B * ROWS_PER_BATCH, LANES), f32),
            jax.ShapeDtypeStruct((B * ROWS_PER_BATCH, LANES), f32),
            jax.ShapeDtypeStruct((2, 32), f32),
            jax.ShapeDtypeStruct((2, 32), jnp.int32),
        ],
    )(*a_in, *r_in, *smem_arrays)

    loss = pl.pallas_call(
        _loss_kernel,
        in_specs=[
            pl.BlockSpec(memory_space=pltpu.VMEM),
            pl.BlockSpec(memory_space=pltpu.VMEM),
            sspec, sspec,
        ],
        out_specs=pl.BlockSpec(memory_space=pltpu.VMEM),
        out_shape=jax.ShapeDtypeStruct((1, 1), f32),
    )(l5, posb, colmax, colarg)
    return loss.reshape(1)


# speedup vs baseline: 33.5712x; 1.0809x over previous
import jax
import jax.numpy as jnp
from jax.experimental import pallas as pl
from jax.experimental.pallas import tpu as pltpu

MD_THRES = 0.5
BETA = 1.0 / 9.0
PI180 = 3.14159265358979323846 / 180.0

LANES = 128
SUBL = 8
TILE = LANES * SUBL
NB = 5
NPAD = TILE * NB
ROWS_PER_BATCH = NPAD // LANES


def _batcher_net(n):
    pairs = []

    def merge(lo, cnt, r):
        step = r * 2
        if step < cnt:
            merge(lo, cnt, step)
            merge(lo + r, cnt, step)
            for i in range(lo + r, lo + cnt - r, step):
                pairs.append((i, i + r))
        else:
            pairs.append((lo, lo + r))

    def sort(lo, cnt):
        if cnt > 1:
            m = cnt // 2
            sort(lo, m)
            sort(lo + m, m)
            merge(lo, cnt, 1)

    p2 = 1 << (n - 1).bit_length()
    sort(0, p2)
    return [(a, b) for (a, b) in pairs if a < n and b < n]


NET24 = _batcher_net(24)


def _rmax2(x):
    return jnp.max(jnp.max(x, axis=1, keepdims=True), axis=0, keepdims=True)


def _rmin2(x):
    return jnp.min(jnp.min(x, axis=1, keepdims=True), axis=0, keepdims=True)


def _rsum2(x):
    return jnp.sum(jnp.sum(x, axis=1, keepdims=True), axis=0, keepdims=True)


def _pairs_kernel(acx_ref, acy_ref, aw_ref, ah_ref, at_ref,
                  r0_ref, r1_ref, r2_ref, r3_ref, r4_ref,
                  gcx_ref, gcy_ref, gw_ref, gh_ref, gtan_ref,
                  sgx0_ref, sgy0_ref, sgx1_ref, sgy1_ref, gasq_ref,
                  cbx0_ref, cbx1_ref, cbx2_ref, cbx3_ref,
                  cby0_ref, cby1_ref, cby2_ref, cby3_ref,
                  gab_ref,
                  l5_ref, posb_ref, colmax_ref, colarg_ref):
    g = pl.program_id(0)
    j = g // NB
    lb = g % NB

    f32 = jnp.float32
    acx = acx_ref[:, :]
    acy = acy_ref[:, :]
    aw = aw_ref[:, :]
    ah = ah_ref[:, :]
    at = at_ref[:, :]

    rowi = jax.lax.broadcasted_iota(jnp.int32, (SUBL, LANES), 0)
    lane = jax.lax.broadcasted_iota(jnp.int32, (SUBL, LANES), 1)
    lidx = lb * TILE + rowi * LANES + lane
    valid = lidx < 5000

    s_a = jnp.maximum(aw, ah)
    ax0 = acx - s_a / 2
    ay0 = acy - s_a / 2
    ax1 = acx + s_a / 2
    ay1 = acy + s_a / 2
    area_asq = (ax1 - ax0) * (ay1 - ay0)
    area_a = aw * ah

    aa = at * PI180
    ca = jnp.cos(aa)
    sa = jnp.sin(aa)
    tan_a = jnp.tan(aa)
    hw = aw / 2
    hh = ah / 2
    DXS = (-1.0, 1.0, 1.0, -1.0)
    DYS = (-1.0, -1.0, 1.0, 1.0)
    AX = [acx + (dx * hw) * ca - (dy * hh) * sa for dx, dy in zip(DXS, DYS)]
    AY = [acy + (dx * hw) * sa + (dy * hh) * ca for dx, dy in zip(DXS, DYS)]
    EAX = [AX[(k + 1) % 4] - AX[k] for k in range(4)]
    EAY = [AY[(k + 1) % 4] - AY[k] for k in range(4)]

    @pl.when(g == 0)
    def _init():
        colmax_ref[:, :] = jnp.full((2, 32), -1.0, f32)
        colarg_ref[:, :] = jnp.zeros((2, 32), jnp.int32)

    cmv0 = colmax_ref[pl.ds(j, 1), :]
    cav0 = colarg_ref[pl.ds(j, 1), :]

    iota32 = jax.lax.broadcasted_iota(jnp.int32, (1, 32), 1)

    def compute_md(m):
        ltx = jnp.maximum(ax0, sgx0_ref[j, m])
        lty = jnp.maximum(ay0, sgy0_ref[j, m])
        rbx = jnp.minimum(ax1, sgx1_ref[j, m])
        rby = jnp.minimum(ay1, sgy1_ref[j, m])
        iw = jnp.clip(rbx - ltx, 0.0, None)
        ih = jnp.clip(rby - lty, 0.0, None)
        inter_sq = iw * ih
        union_sq = area_asq + gasq_ref[j, m] - inter_sq
        bf = inter_sq / jnp.maximum(union_sq, 1e-9)

        BX = (cbx0_ref[j, m], cbx1_ref[j, m], cbx2_ref[j, m], cbx3_ref[j, m])
        BY = (cby0_ref[j, m], cby1_ref[j, m], cby2_ref[j, m], cby3_ref[j, m])
        EBX = [BX[(k + 1) % 4] - BX[k] for k in range(4)]
        EBY = [BY[(k + 1) % 4] - BY[k] for k in range(4)]

        pts_x, pts_y, mf = [], [], []
        for p in range(4):
            ok = None
            for k in range(4):
                cr = EBX[k] * (AY[p] - BY[k]) - EBY[k] * (AX[p] - BX[k])
                c = cr >= -1e-9
                ok = c if ok is None else (ok & c)
            pts_x.append(AX[p])
            pts_y.append(AY[p])
            mf.append(ok)
        for q in range(4):
            ok = None
            for k in range(4):
                cr = EAX[k] * (jnp.float32(BY[q]) - AY[k]) - EAY[k] * (jnp.float32(BX[q]) - AX[k])
                c = cr >= -1e-9
                ok = c if ok is None else (ok & c)
            pts_x.append(jnp.full((SUBL, LANES), BX[q], f32))
            pts_y.append(jnp.full((SUBL, LANES), BY[q], f32))
            mf.append(ok)
        for p in range(4):
            for q in range(4):
                rx, ry = EAX[p], EAY[p]
                sx, sy = EBX[q], EBY[q]
                qpx = BX[q] - AX[p]
                qpy = BY[q] - AY[p]
                denom = rx * sy - ry * sx
                okd = jnp.abs(denom) > 1e-12
                den = jnp.where(okd, denom, 1.0)
                t = (qpx * sy - qpy * sx) / den
                u = (qpx * ry - qpy * rx) / den
                vv = okd & (t >= 0.0) & (t <= 1.0) & (u >= 0.0) & (u <= 1.0)
                pts_x.append(AX[p] + t * rx)
                pts_y.append(AY[p] + t * ry)
                mf.append(vv)

        mflt = [jnp.where(mm, 1.0, 0.0) for mm in mf]
        cnt = mflt[0]
        for k in range(1, 24):
            cnt = cnt + mflt[k]
        cntc = jnp.maximum(cnt, 1.0)
        ctrx = pts_x[0] * mflt[0]
        ctry = pts_y[0] * mflt[0]
        for k in range(1, 24):
            ctrx = ctrx + pts_x[k] * mflt[k]
            ctry = ctry + pts_y[k] * mflt[k]
        ctrx = ctrx / cntc
        ctry = ctry / cntc

        K = []
        for k in range(24):
            dx = pts_x[k] - ctrx
            dy = pts_y[k] - ctry
            sden = jnp.abs(dx) + jnp.abs(dy)
            r = dx / jnp.where(sden == 0.0, 1.0, sden)
            key = jnp.where(dy >= 0.0, 1.0 - r, r - 1.0)
            K.append(jnp.where(mf[k], key, 1e9))
        X = list(pts_x)
        Y = list(pts_y)
        for a, b in NET24:
            sw = K[a] > K[b]
            ka = jnp.where(sw, K[b], K[a])
            kb = jnp.where(sw, K[a], K[b])
            xa = jnp.where(sw, X[b], X[a])
            xb = jnp.where(sw, X[a], X[b])
            ya = jnp.where(sw, Y[b], Y[a])
            yb = jnp.where(sw, Y[a], Y[b])
            K[a], K[b], X[a], X[b], Y[a], Y[b] = ka, kb, xa, xb, ya, yb
        PX = [jnp.where(cnt > k, X[k], X[0]) for k in range(24)]
        PY = [jnp.where(cnt > k, Y[k], Y[0]) for k in range(24)]
        crs = PX[23] * PY[0] - PY[23] * PX[0]
        for k in range(23):
            crs = crs + (PX[k] * PY[k + 1] - PY[k] * PX[k + 1])
        area = 0.5 * jnp.abs(crs)
        inter = jnp.where(cnt >= 3.0, area, 0.0)
        iou = inter / jnp.maximum(area_a + gab_ref[j, m] - inter, 1e-9)
        return jnp.where(bf > 0.1, iou, 0.0)

    def gt_body(i, carry):
        rmax, rarg, cmv, cav = carry
        mds = [compute_md(8 * i + t) for t in range(8)]
        for t in range(8):
            m = 8 * i + t
            md = mds[t]
            upd = md > rmax
            rmax = jnp.where(upd, md, rmax)
            rarg = jnp.where(upd, m, rarg)
            mdc = jnp.where(valid, md, -1.0)
            mx = _rmax2(mdc)
            mxb8 = jnp.broadcast_to(mx, (SUBL, LANES))
            cand = _rmin2(jnp.where(mdc == mxb8, lidx, jnp.int32(2 ** 30)))
            mxb = jnp.broadcast_to(mx, (1, 32))
            cnb = jnp.broadcast_to(cand, (1, 32))
            better = (iota32 == m) & (mxb > cmv)
            cmv = jnp.where(better, mxb, cmv)
            cav = jnp.where(better, cnb, cav)
        return rmax, rarg, cmv, cav

    rmax0 = jnp.full((SUBL, LANES), -1.0, f32)
    rarg0 = jnp.zeros((SUBL, LANES), jnp.int32)
    rmax, rarg, cmv, cav = jax.lax.fori_loop(
        0, 4, gt_body, (rmax0, rarg0, cmv0, cav0))

    colmax_ref[pl.ds(j, 1), :] = cmv
    colarg_ref[pl.ds(j, 1), :] = cav

    g0x = jnp.full((SUBL, LANES), gcx_ref[j, 0], f32)
    g0y = jnp.full((SUBL, LANES), gcy_ref[j, 0], f32)
    g0w = jnp.full((SUBL, LANES), gw_ref[j, 0], f32)
    g0h = jnp.full((SUBL, LANES), gh_ref[j, 0], f32)
    g0t = jnp.full((SUBL, LANES), gtan_ref[j, 0], f32)
    for m in range(1, 32):
        sel = rarg == m
        g0x = jnp.where(sel, gcx_ref[j, m], g0x)
        g0y = jnp.where(sel, gcy_ref[j, m], g0y)
        g0w = jnp.where(sel, gw_ref[j, m], g0w)
        g0h = jnp.where(sel, gh_ref[j, m], g0h)
        g0t = jnp.where(sel, gtan_ref[j, m], g0t)

    d0 = 10.0 * (g0x - acx) / aw
    d1 = 10.0 * (g0y - acy) / ah
    d2 = 5.0 * jnp.log(g0w / aw)
    d3 = 5.0 * jnp.log(g0h / ah)
    d4 = 15.0 * (g0t - tan_a)
    l5 = jnp.zeros((SUBL, LANES), f32)
    for t_, r_ in ((d0, r0_ref), (d1, r1_ref), (d2, r2_ref),
                   (d3, r3_ref), (d4, r4_ref)):
        diff = jnp.abs(r_[:, :] - t_)
        l5 = l5 + jnp.where(diff < BETA, 0.5 * diff * diff / BETA,
                            diff - 0.5 * BETA)

    l5_ref[:, :] = jnp.where(valid, l5, 0.0)
    posb_ref[:, :] = jnp.where(valid & (rmax >= MD_THRES), 1.0, 0.0)


def _loss_kernel(l5_ref, posb_ref, colmax_ref, colarg_ref, out_ref):
    R = ROWS_PER_BATCH
    rowi = jax.lax.broadcasted_iota(jnp.int32, (R, LANES), 0)
    lane = jax.lax.broadcasted_iota(jnp.int32, (R, LANES), 1)
    idx = rowi * LANES + lane
    total = jnp.zeros((1, 1), jnp.float32)
    for j in range(2):
        l5 = l5_ref[pl.ds(j * R, R), :]
        pos = posb_ref[pl.ds(j * R, R), :]
        for m in range(32):
            am = colarg_ref[j, m]
            fm = jnp.where(colmax_ref[j, m] < MD_THRES, 1.0, 0.0)
            pos = jnp.maximum(pos, jnp.where(idx == am, fm, 0.0))
        S = _rsum2(pos * l5)
        num = jnp.maximum(_rsum2(pos), 1.0)
        total = total + S / (num * 5.0)
    out_ref[:, :] = total * 0.5


def kernel(regressions, anchors, refined_achors, annotations):
    f32 = jnp.float32
    B, N, _ = anchors.shape

    def acomp(x, c, pad):
        v = x[:, :, c]
        v = jnp.pad(v, ((0, 0), (0, NPAD - N)), constant_values=pad)
        return v.reshape(B * ROWS_PER_BATCH, LANES)

    a_in = [acomp(anchors, c, 1.0) for c in range(5)]
    r_in = [acomp(regressions, c, 0.0) for c in range(5)]

    gcx = annotations[:, :, 0]
    gcy = annotations[:, :, 1]
    gw = annotations[:, :, 2]
    gh = annotations[:, :, 3]
    gt = annotations[:, :, 4]
    s_g = jnp.maximum(gw, gh)
    sgx0 = gcx - s_g / 2
    sgy0 = gcy - s_g / 2
    sgx1 = gcx + s_g / 2
    sgy1 = gcy + s_g / 2
    gasq = (sgx1 - sgx0) * (sgy1 - sgy0)
    ga = gt * (jnp.pi / 180.0)
    cg, sg = jnp.cos(ga), jnp.sin(ga)
    gtan = jnp.tan(ga)
    cbx = []
    cby = []
    for dx, dy in ((-0.5, -0.5), (0.5, -0.5), (0.5, 0.5), (-0.5, 0.5)):
        cbx.append(gcx + (dx * gw) * cg - (dy * gh) * sg)
        cby.append(gcy + (dx * gw) * sg + (dy * gh) * cg)
    gab = gw * gh

    smem_arrays = [gcx, gcy, gw, gh, gtan, sgx0, sgy0, sgx1, sgy1, gasq,
                   cbx[0], cbx[1], cbx[2], cbx[3],
                   cby[0], cby[1], cby[2], cby[3], gab]

    vspec = pl.BlockSpec((SUBL, LANES), lambda g: (g, 0))
    sspec = pl.BlockSpec(memory_space=pltpu.SMEM)
    fullspec = pl.BlockSpec((2, 32), lambda g: (0, 0))

    l5, posb, colmax, colarg = pl.pallas_call(
        _pairs_kernel,
        grid=(B * NB,),
        in_specs=[vspec] * 10 + [sspec] * 19,
        out_specs=[
            vspec, vspec, fullspec, fullspec,
        ],
        out_shape=[
            jax.ShapeDtypeStruct((B * ROWS_PER_BATCH, LANES), f32),
            jax.ShapeDtypeStruct((B * ROWS_PER_BATCH, LANES), f32),
            jax.ShapeDtypeStruct((2, 32), f32),
            jax.ShapeDtypeStruct((2, 32), jnp.int32),
        ],
    )(*a_in, *r_in, *smem_arrays)

    loss = pl.pallas_call(
        _loss_kernel,
        in_specs=[
            pl.BlockSpec(memory_space=pltpu.VMEM),
            pl.BlockSpec(memory_space=pltpu.VMEM),
            sspec, sspec,
        ],
        out_specs=pl.BlockSpec(memory_space=pltpu.VMEM),
        out_shape=jax.ShapeDtypeStruct((1, 1), f32),
    )(l5, posb, colmax, colarg)
    return loss.reshape(1)


# speedup vs baseline: 35.0429x; 1.0438x over previous
import jax
import jax.numpy as jnp
from jax.experimental import pallas as pl
from jax.experimental.pallas import tpu as pltpu

MD_THRES = 0.5
BETA = 1.0 / 9.0
PI180 = 3.14159265358979323846 / 180.0

LANES = 128
SUBL = 8
TILE = LANES * SUBL
NB = 5
NPAD = TILE * NB
ROWS_PER_BATCH = NPAD // LANES


def _batcher_net(n):
    pairs = []

    def merge(lo, cnt, r):
        step = r * 2
        if step < cnt:
            merge(lo, cnt, step)
            merge(lo + r, cnt, step)
            for i in range(lo + r, lo + cnt - r, step):
                pairs.append((i, i + r))
        else:
            pairs.append((lo, lo + r))

    def sort(lo, cnt):
        if cnt > 1:
            m = cnt // 2
            sort(lo, m)
            sort(lo + m, m)
            merge(lo, cnt, 1)

    p2 = 1 << (n - 1).bit_length()
    sort(0, p2)
    return [(a, b) for (a, b) in pairs if a < n and b < n]


NET24 = _batcher_net(24)


def _rmax2(x):
    return jnp.max(jnp.max(x, axis=1, keepdims=True), axis=0, keepdims=True)


def _rmin2(x):
    return jnp.min(jnp.min(x, axis=1, keepdims=True), axis=0, keepdims=True)


def _rsum2(x):
    return jnp.sum(jnp.sum(x, axis=1, keepdims=True), axis=0, keepdims=True)


def _pairs_kernel(acx_ref, acy_ref, aw_ref, ah_ref, at_ref,
                  r0_ref, r1_ref, r2_ref, r3_ref, r4_ref,
                  gcx_ref, gcy_ref, gw_ref, gh_ref, gtan_ref,
                  sgx0_ref, sgy0_ref, sgx1_ref, sgy1_ref, gasq_ref,
                  cbx0_ref, cbx1_ref, cbx2_ref, cbx3_ref,
                  cby0_ref, cby1_ref, cby2_ref, cby3_ref,
                  gab_ref,
                  l5_ref, posb_ref, colmax_ref, colarg_ref):
    g = pl.program_id(0)
    j = g // NB
    lb = g % NB

    f32 = jnp.float32
    acx = acx_ref[:, :]
    acy = acy_ref[:, :]
    aw = aw_ref[:, :]
    ah = ah_ref[:, :]
    at = at_ref[:, :]

    rowi = jax.lax.broadcasted_iota(jnp.int32, (SUBL, LANES), 0)
    lane = jax.lax.broadcasted_iota(jnp.int32, (SUBL, LANES), 1)
    lidx = lb * TILE + rowi * LANES + lane
    valid = lidx < 5000

    s_a = jnp.maximum(aw, ah)
    ax0 = acx - s_a / 2
    ay0 = acy - s_a / 2
    ax1 = acx + s_a / 2
    ay1 = acy + s_a / 2
    area_asq = (ax1 - ax0) * (ay1 - ay0)
    area_a = aw * ah

    aa = at * PI180
    ca = jnp.cos(aa)
    sa = jnp.sin(aa)
    tan_a = jnp.tan(aa)
    hw = aw / 2
    hh = ah / 2
    DXS = (-1.0, 1.0, 1.0, -1.0)
    DYS = (-1.0, -1.0, 1.0, 1.0)
    AX = [acx + (dx * hw) * ca - (dy * hh) * sa for dx, dy in zip(DXS, DYS)]
    AY = [acy + (dx * hw) * sa + (dy * hh) * ca for dx, dy in zip(DXS, DYS)]
    EAX = [AX[(k + 1) % 4] - AX[k] for k in range(4)]
    EAY = [AY[(k + 1) % 4] - AY[k] for k in range(4)]

    @pl.when(g == 0)
    def _init():
        colmax_ref[:, :] = jnp.full((2, 32), -1.0, f32)
        colarg_ref[:, :] = jnp.zeros((2, 32), jnp.int32)

    cmv0 = colmax_ref[pl.ds(j, 1), :]
    cav0 = colarg_ref[pl.ds(j, 1), :]

    iota32 = jax.lax.broadcasted_iota(jnp.int32, (1, 32), 1)

    def compute_md(m):
        ltx = jnp.maximum(ax0, sgx0_ref[j, m])
        lty = jnp.maximum(ay0, sgy0_ref[j, m])
        rbx = jnp.minimum(ax1, sgx1_ref[j, m])
        rby = jnp.minimum(ay1, sgy1_ref[j, m])
        iw = jnp.clip(rbx - ltx, 0.0, None)
        ih = jnp.clip(rby - lty, 0.0, None)
        inter_sq = iw * ih
        union_sq = area_asq + gasq_ref[j, m] - inter_sq
        bf = inter_sq / jnp.maximum(union_sq, 1e-9)

        BX = (cbx0_ref[j, m], cbx1_ref[j, m], cbx2_ref[j, m], cbx3_ref[j, m])
        BY = (cby0_ref[j, m], cby1_ref[j, m], cby2_ref[j, m], cby3_ref[j, m])
        EBX = [BX[(k + 1) % 4] - BX[k] for k in range(4)]
        EBY = [BY[(k + 1) % 4] - BY[k] for k in range(4)]

        pts_x, pts_y, mf = [], [], []
        for p in range(4):
            ok = None
            for k in range(4):
                cr = EBX[k] * (AY[p] - BY[k]) - EBY[k] * (AX[p] - BX[k])
                c = cr >= -1e-9
                ok = c if ok is None else (ok & c)
            pts_x.append(AX[p])
            pts_y.append(AY[p])
            mf.append(ok)
        for q in range(4):
            ok = None
            for k in range(4):
                cr = EAX[k] * (jnp.float32(BY[q]) - AY[k]) - EAY[k] * (jnp.float32(BX[q]) - AX[k])
                c = cr >= -1e-9
                ok = c if ok is None else (ok & c)
            pts_x.append(jnp.full((SUBL, LANES), BX[q], f32))
            pts_y.append(jnp.full((SUBL, LANES), BY[q], f32))
            mf.append(ok)
        for p in range(4):
            for q in range(4):
                rx, ry = EAX[p], EAY[p]
                sx, sy = EBX[q], EBY[q]
                qpx = BX[q] - AX[p]
                qpy = BY[q] - AY[p]
                denom = rx * sy - ry * sx
                okd = jnp.abs(denom) > 1e-12
                den = jnp.where(okd, denom, 1.0)
                t = (qpx * sy - qpy * sx) / den
                u = (qpx * ry - qpy * rx) / den
                vv = okd & (t >= 0.0) & (t <= 1.0) & (u >= 0.0) & (u <= 1.0)
                pts_x.append(AX[p] + t * rx)
                pts_y.append(AY[p] + t * ry)
                mf.append(vv)

        mflt = [jnp.where(mm, 1.0, 0.0) for mm in mf]
        cnt = mflt[0]
        for k in range(1, 24):
            cnt = cnt + mflt[k]
        cntc = jnp.maximum(cnt, 1.0)
        ctrx = pts_x[0] * mflt[0]
        ctry = pts_y[0] * mflt[0]
        for k in range(1, 24):
            ctrx = ctrx + pts_x[k] * mflt[k]
            ctry = ctry + pts_y[k] * mflt[k]
        ctrx = ctrx / cntc
        ctry = ctry / cntc

        K = []
        for k in range(24):
            dx = pts_x[k] - ctrx
            dy = pts_y[k] - ctry
            sden = jnp.abs(dx) + jnp.abs(dy)
            r = dx / jnp.where(sden == 0.0, 1.0, sden)
            key = jnp.where(dy >= 0.0, 1.0 - r, r - 1.0)
            K.append(jnp.where(mf[k], key, 1e9))
        X = list(pts_x)
        Y = list(pts_y)
        for a, b in NET24:
            sw = K[a] > K[b]
            ka = jnp.where(sw, K[b], K[a])
            kb = jnp.where(sw, K[a], K[b])
            xa = jnp.where(sw, X[b], X[a])
            xb = jnp.where(sw, X[a], X[b])
            ya = jnp.where(sw, Y[b], Y[a])
            yb = jnp.where(sw, Y[a], Y[b])
            K[a], K[b], X[a], X[b], Y[a], Y[b] = ka, kb, xa, xb, ya, yb
        PX = [jnp.where(cnt > k, X[k], X[0]) for k in range(24)]
        PY = [jnp.where(cnt > k, Y[k], Y[0]) for k in range(24)]
        crs = PX[23] * PY[0] - PY[23] * PX[0]
        for k in range(23):
            crs = crs + (PX[k] * PY[k + 1] - PY[k] * PX[k + 1])
        area = 0.5 * jnp.abs(crs)
        inter = jnp.where(cnt >= 3.0, area, 0.0)
        iou = inter / jnp.maximum(area_a + gab_ref[j, m] - inter, 1e-9)
        return jnp.where(bf > 0.1, iou, 0.0)

    def gt_body(i, carry):
        rmax, rarg, cmv, cav = carry
        mds = [compute_md(16 * i + t) for t in range(16)]
        for t in range(16):
            m = 16 * i + t
            md = mds[t]
            upd = md > rmax
            rmax = jnp.where(upd, md, rmax)
            rarg = jnp.where(upd, m, rarg)
            mdc = jnp.where(valid, md, -1.0)
            mx = _rmax2(mdc)
            mxb8 = jnp.broadcast_to(mx, (SUBL, LANES))
            cand = _rmin2(jnp.where(mdc == mxb8, lidx, jnp.int32(2 ** 30)))
            mxb = jnp.broadcast_to(mx, (1, 32))
            cnb = jnp.broadcast_to(cand, (1, 32))
            better = (iota32 == m) & (mxb > cmv)
            cmv = jnp.where(better, mxb, cmv)
            cav = jnp.where(better, cnb, cav)
        return rmax, rarg, cmv, cav

    rmax0 = jnp.full((SUBL, LANES), -1.0, f32)
    rarg0 = jnp.zeros((SUBL, LANES), jnp.int32)
    rmax, rarg, cmv, cav = jax.lax.fori_loop(
        0, 2, gt_body, (rmax0, rarg0, cmv0, cav0))

    colmax_ref[pl.ds(j, 1), :] = cmv
    colarg_ref[pl.ds(j, 1), :] = cav

    g0x = jnp.full((SUBL, LANES), gcx_ref[j, 0], f32)
    g0y = jnp.full((SUBL, LANES), gcy_ref[j, 0], f32)
    g0w = jnp.full((SUBL, LANES), gw_ref[j, 0], f32)
    g0h = jnp.full((SUBL, LANES), gh_ref[j, 0], f32)
    g0t = jnp.full((SUBL, LANES), gtan_ref[j, 0], f32)
    for m in range(1, 32):
        sel = rarg == m
        g0x = jnp.where(sel, gcx_ref[j, m], g0x)
        g0y = jnp.where(sel, gcy_ref[j, m], g0y)
        g0w = jnp.where(sel, gw_ref[j, m], g0w)
        g0h = jnp.where(sel, gh_ref[j, m], g0h)
        g0t = jnp.where(sel, gtan_ref[j, m], g0t)

    d0 = 10.0 * (g0x - acx) / aw
    d1 = 10.0 * (g0y - acy) / ah
    d2 = 5.0 * jnp.log(g0w / aw)
    d3 = 5.0 * jnp.log(g0h / ah)
    d4 = 15.0 * (g0t - tan_a)
    l5 = jnp.zeros((SUBL, LANES), f32)
    for t_, r_ in ((d0, r0_ref), (d1, r1_ref), (d2, r2_ref),
                   (d3, r3_ref), (d4, r4_ref)):
        diff = jnp.abs(r_[:, :] - t_)
        l5 = l5 + jnp.where(diff < BETA, 0.5 * diff * diff / BETA,
                            diff - 0.5 * BETA)

    l5_ref[:, :] = jnp.where(valid, l5, 0.0)
    posb_ref[:, :] = jnp.where(valid & (rmax >= MD_THRES), 1.0, 0.0)


def _loss_kernel(l5_ref, posb_ref, colmax_ref, colarg_ref, out_ref):
    R = ROWS_PER_BATCH
    rowi = jax.lax.broadcasted_iota(jnp.int32, (R, LANES), 0)
    lane = jax.lax.broadcasted_iota(jnp.int32, (R, LANES), 1)
    idx = rowi * LANES + lane
    total = jnp.zeros((1, 1), jnp.float32)
    for j in range(2):
        l5 = l5_ref[pl.ds(j * R, R), :]
        pos = posb_ref[pl.ds(j * R, R), :]
        for m in range(32):
            am = colarg_ref[j, m]
            fm = jnp.where(colmax_ref[j, m] < MD_THRES, 1.0, 0.0)
            pos = jnp.maximum(pos, jnp.where(idx == am, fm, 0.0))
        S = _rsum2(pos * l5)
        num = jnp.maximum(_rsum2(pos), 1.0)
        total = total + S / (num * 5.0)
    out_ref[:, :] = total * 0.5


def kernel(regressions, anchors, refined_achors, annotations):
    f32 = jnp.float32
    B, N, _ = anchors.shape

    def acomp(x, c, pad):
        v = x[:, :, c]
        v = jnp.pad(v, ((0, 0), (0, NPAD - N)), constant_values=pad)
        return v.reshape(B * ROWS_PER_BATCH, LANES)

    a_in = [acomp(anchors, c, 1.0) for c in range(5)]
    r_in = [acomp(regressions, c, 0.0) for c in range(5)]

    gcx = annotations[:, :, 0]
    gcy = annotations[:, :, 1]
    gw = annotations[:, :, 2]
    gh = annotations[:, :, 3]
    gt = annotations[:, :, 4]
    s_g = jnp.maximum(gw, gh)
    sgx0 = gcx - s_g / 2
    sgy0 = gcy - s_g / 2
    sgx1 = gcx + s_g / 2
    sgy1 = gcy + s_g / 2
    gasq = (sgx1 - sgx0) * (sgy1 - sgy0)
    ga = gt * (jnp.pi / 180.0)
    cg, sg = jnp.cos(ga), jnp.sin(ga)
    gtan = jnp.tan(ga)
    cbx = []
    cby = []
    for dx, dy in ((-0.5, -0.5), (0.5, -0.5), (0.5, 0.5), (-0.5, 0.5)):
        cbx.append(gcx + (dx * gw) * cg - (dy * gh) * sg)
        cby.append(gcy + (dx * gw) * sg + (dy * gh) * cg)
    gab = gw * gh

    smem_arrays = [gcx, gcy, gw, gh, gtan, sgx0, sgy0, sgx1, sgy1, gasq,
                   cbx[0], cbx[1], cbx[2], cbx[3],
                   cby[0], cby[1], cby[2], cby[3], gab]

    vspec = pl.BlockSpec((SUBL, LANES), lambda g: (g, 0))
    sspec = pl.BlockSpec(memory_space=pltpu.SMEM)
    fullspec = pl.BlockSpec((2, 32), lambda g: (0, 0))

    l5, posb, colmax, colarg = pl.pallas_call(
        _pairs_kernel,
        grid=(B * NB,),
        in_specs=[vspec] * 10 + [sspec] * 19,
        out_specs=[
            vspec, vspec, fullspec, fullspec,
        ],
        out_shape=[
            jax.ShapeDtypeStruct((B * ROWS_PER_BATCH, LANES), f32),
            jax.ShapeDtypeStruct((B * ROWS_PER_BATCH, LANES), f32),
            jax.ShapeDtypeStruct((2, 32), f32),
            jax.ShapeDtypeStruct((2, 32), jnp.int32),
        ],
    )(*a_in, *r_in, *smem_arrays)

    loss = pl.pallas_call(
        _loss_kernel,
        in_specs=[
            pl.BlockSpec(memory_space=pltpu.VMEM),
            pl.BlockSpec(memory_space=pltpu.VMEM),
            sspec, sspec,
        ],
        out_specs=pl.BlockSpec(memory_space=pltpu.VMEM),
        out_shape=jax.ShapeDtypeStruct((1, 1), f32),
    )(l5, posb, colmax, colarg)
    return loss.reshape(1)


# speedup vs baseline: 37.1128x; 1.0591x over previous
import jax
import jax.numpy as jnp
from jax.experimental import pallas as pl
from jax.experimental.pallas import tpu as pltpu

MD_THRES = 0.5
BETA = 1.0 / 9.0
PI180 = 3.14159265358979323846 / 180.0

LANES = 128
SUBL = 8
TILE = LANES * SUBL
NB = 5
NPAD = TILE * NB
ROWS_PER_BATCH = NPAD // LANES


def _batcher_net(n):
    pairs = []

    def merge(lo, cnt, r):
        step = r * 2
        if step < cnt:
            merge(lo, cnt, step)
            merge(lo + r, cnt, step)
            for i in range(lo + r, lo + cnt - r, step):
                pairs.append((i, i + r))
        else:
            pairs.append((lo, lo + r))

    def sort(lo, cnt):
        if cnt > 1:
            m = cnt // 2
            sort(lo, m)
            sort(lo + m, m)
            merge(lo, cnt, 1)

    p2 = 1 << (n - 1).bit_length()
    sort(0, p2)
    return [(a, b) for (a, b) in pairs if a < n and b < n]


NET24 = _batcher_net(24)


def _rmax2(x):
    return jnp.max(jnp.max(x, axis=1, keepdims=True), axis=0, keepdims=True)


def _rmin2(x):
    return jnp.min(jnp.min(x, axis=1, keepdims=True), axis=0, keepdims=True)


def _rsum2(x):
    return jnp.sum(jnp.sum(x, axis=1, keepdims=True), axis=0, keepdims=True)


def _pairs_kernel(acx_ref, acy_ref, aw_ref, ah_ref, at_ref,
                  r0_ref, r1_ref, r2_ref, r3_ref, r4_ref,
                  gcx_ref, gcy_ref, gw_ref, gh_ref, gtan_ref,
                  sgx0_ref, sgy0_ref, sgx1_ref, sgy1_ref, gasq_ref,
                  cbx0_ref, cbx1_ref, cbx2_ref, cbx3_ref,
                  cby0_ref, cby1_ref, cby2_ref, cby3_ref,
                  gab_ref,
                  l5_ref, posb_ref, colmax_ref, colarg_ref):
    g = pl.program_id(0)
    j = g // NB
    lb = g % NB

    f32 = jnp.float32
    acx = acx_ref[:, :]
    acy = acy_ref[:, :]
    aw = aw_ref[:, :]
    ah = ah_ref[:, :]
    at = at_ref[:, :]

    rowi = jax.lax.broadcasted_iota(jnp.int32, (SUBL, LANES), 0)
    lane = jax.lax.broadcasted_iota(jnp.int32, (SUBL, LANES), 1)
    lidx = lb * TILE + rowi * LANES + lane
    valid = lidx < 5000

    s_a = jnp.maximum(aw, ah)
    ax0 = acx - s_a / 2
    ay0 = acy - s_a / 2
    ax1 = acx + s_a / 2
    ay1 = acy + s_a / 2
    area_asq = (ax1 - ax0) * (ay1 - ay0)
    area_a = aw * ah

    aa = at * PI180
    ca = jnp.cos(aa)
    sa = jnp.sin(aa)
    tan_a = jnp.tan(aa)
    hw = aw / 2
    hh = ah / 2
    DXS = (-1.0, 1.0, 1.0, -1.0)
    DYS = (-1.0, -1.0, 1.0, 1.0)
    AX = [acx + (dx * hw) * ca - (dy * hh) * sa for dx, dy in zip(DXS, DYS)]
    AY = [acy + (dx * hw) * sa + (dy * hh) * ca for dx, dy in zip(DXS, DYS)]
    EAX = [AX[(k + 1) % 4] - AX[k] for k in range(4)]
    EAY = [AY[(k + 1) % 4] - AY[k] for k in range(4)]

    @pl.when(g == 0)
    def _init():
        colmax_ref[:, :] = jnp.full((2, 32), -1.0, f32)
        colarg_ref[:, :] = jnp.zeros((2, 32), jnp.int32)

    cmv0 = colmax_ref[pl.ds(j, 1), :]
    cav0 = colarg_ref[pl.ds(j, 1), :]

    iota32 = jax.lax.broadcasted_iota(jnp.int32, (1, 32), 1)

    def compute_md(m):
        ltx = jnp.maximum(ax0, sgx0_ref[j, m])
        lty = jnp.maximum(ay0, sgy0_ref[j, m])
        rbx = jnp.minimum(ax1, sgx1_ref[j, m])
        rby = jnp.minimum(ay1, sgy1_ref[j, m])
        iw = jnp.clip(rbx - ltx, 0.0, None)
        ih = jnp.clip(rby - lty, 0.0, None)
        inter_sq = iw * ih
        union_sq = area_asq + gasq_ref[j, m] - inter_sq
        bf = inter_sq / jnp.maximum(union_sq, 1e-9)

        BX = (cbx0_ref[j, m], cbx1_ref[j, m], cbx2_ref[j, m], cbx3_ref[j, m])
        BY = (cby0_ref[j, m], cby1_ref[j, m], cby2_ref[j, m], cby3_ref[j, m])
        EBX = [BX[(k + 1) % 4] - BX[k] for k in range(4)]
        EBY = [BY[(k + 1) % 4] - BY[k] for k in range(4)]

        DX = [[BX[q] - AX[p] for q in range(4)] for p in range(4)]
        DY = [[BY[q] - AY[p] for q in range(4)] for p in range(4)]

        pts_x, pts_y, mf = [], [], []
        for p in range(4):
            ok = None
            for k in range(4):
                cr = EBY[k] * DX[p][k] - EBX[k] * DY[p][k]
                c = cr >= -1e-9
                ok = c if ok is None else (ok & c)
            pts_x.append(AX[p])
            pts_y.append(AY[p])
            mf.append(ok)
        for q in range(4):
            ok = None
            for k in range(4):
                cr = EAX[k] * DY[k][q] - EAY[k] * DX[k][q]
                c = cr >= -1e-9
                ok = c if ok is None else (ok & c)
            pts_x.append(jnp.full((SUBL, LANES), BX[q], f32))
            pts_y.append(jnp.full((SUBL, LANES), BY[q], f32))
            mf.append(ok)
        for p in range(4):
            for q in range(4):
                rx, ry = EAX[p], EAY[p]
                sx, sy = EBX[q], EBY[q]
                qpx = DX[p][q]
                qpy = DY[p][q]
                denom = rx * sy - ry * sx
                okd = jnp.abs(denom) > 1e-12
                den = jnp.where(okd, denom, 1.0)
                t = (qpx * sy - qpy * sx) / den
                u = (qpx * ry - qpy * rx) / den
                vv = okd & (t >= 0.0) & (t <= 1.0) & (u >= 0.0) & (u <= 1.0)
                pts_x.append(AX[p] + t * rx)
                pts_y.append(AY[p] + t * ry)
                mf.append(vv)

        mflt = [jnp.where(mm, 1.0, 0.0) for mm in mf]
        cnt = mflt[0]
        for k in range(1, 24):
            cnt = cnt + mflt[k]
        cntc = jnp.maximum(cnt, 1.0)
        ctrx = pts_x[0] * mflt[0]
        ctry = pts_y[0] * mflt[0]
        for k in range(1, 24):
            ctrx = ctrx + pts_x[k] * mflt[k]
            ctry = ctry + pts_y[k] * mflt[k]
        ctrx = ctrx / cntc
        ctry = ctry / cntc

        K = []
        for k in range(24):
            dx = pts_x[k] - ctrx
            dy = pts_y[k] - ctry
            sden = jnp.abs(dx) + jnp.abs(dy)
            r = dx / jnp.maximum(sden, 1e-37)
            key = jnp.where(dy >= 0.0, 1.0 - r, r - 1.0)
            K.append(jnp.where(mf[k], key, 1e9))
        X = list(pts_x)
        Y = list(pts_y)
        for a, b in NET24:
            sw = K[a] > K[b]
            ka = jnp.where(sw, K[b], K[a])
            kb = jnp.where(sw, K[a], K[b])
            xa = jnp.where(sw, X[b], X[a])
            xb = jnp.where(sw, X[a], X[b])
            ya = jnp.where(sw, Y[b], Y[a])
            yb = jnp.where(sw, Y[a], Y[b])
            K[a], K[b], X[a], X[b], Y[a], Y[b] = ka, kb, xa, xb, ya, yb
        PX = [jnp.where(cnt > k, X[k], X[0]) for k in range(24)]
        PY = [jnp.where(cnt > k, Y[k], Y[0]) for k in range(24)]
        crs = PX[23] * PY[0] - PY[23] * PX[0]
        for k in range(23):
            crs = crs + (PX[k] * PY[k + 1] - PY[k] * PX[k + 1])
        area = 0.5 * jnp.abs(crs)
        inter = jnp.where(cnt >= 3.0, area, 0.0)
        iou = inter / jnp.maximum(area_a + gab_ref[j, m] - inter, 1e-9)
        return jnp.where(bf > 0.1, iou, 0.0)

    def gt_body(i, carry):
        rmax, rarg, cmv, cav = carry
        mds = [compute_md(16 * i + t) for t in range(16)]
        for t in range(16):
            m = 16 * i + t
            md = mds[t]
            upd = md > rmax
            rmax = jnp.where(upd, md, rmax)
            rarg = jnp.where(upd, m, rarg)
            mdc = jnp.where(valid, md, -1.0)
            mx = _rmax2(mdc)
            mxb8 = jnp.broadcast_to(mx, (SUBL, LANES))
            cand = _rmin2(jnp.where(mdc == mxb8, lidx, jnp.int32(2 ** 30)))
            mxb = jnp.broadcast_to(mx, (1, 32))
            cnb = jnp.broadcast_to(cand, (1, 32))
            better = (iota32 == m) & (mxb > cmv)
            cmv = jnp.where(better, mxb, cmv)
            cav = jnp.where(better, cnb, cav)
        return rmax, rarg, cmv, cav

    rmax0 = jnp.full((SUBL, LANES), -1.0, f32)
    rarg0 = jnp.zeros((SUBL, LANES), jnp.int32)
    rmax, rarg, cmv, cav = jax.lax.fori_loop(
        0, 2, gt_body, (rmax0, rarg0, cmv0, cav0))

    colmax_ref[pl.ds(j, 1), :] = cmv
    colarg_ref[pl.ds(j, 1), :] = cav

    g0x = jnp.full((SUBL, LANES), gcx_ref[j, 0], f32)
    g0y = jnp.full((SUBL, LANES), gcy_ref[j, 0], f32)
    g0w = jnp.full((SUBL, LANES), gw_ref[j, 0], f32)
    g0h = jnp.full((SUBL, LANES), gh_ref[j, 0], f32)
    g0t = jnp.full((SUBL, LANES), gtan_ref[j, 0], f32)
    for m in range(1, 32):
        sel = rarg == m
        g0x = jnp.where(sel, gcx_ref[j, m], g0x)
        g0y = jnp.where(sel, gcy_ref[j, m], g0y)
        g0w = jnp.where(sel, gw_ref[j, m], g0w)
        g0h = jnp.where(sel, gh_ref[j, m], g0h)
        g0t = jnp.where(sel, gtan_ref[j, m], g0t)

    d0 = 10.0 * (g0x - acx) / aw
    d1 = 10.0 * (g0y - acy) / ah
    d2 = 5.0 * jnp.log(g0w / aw)
    d3 = 5.0 * jnp.log(g0h / ah)
    d4 = 15.0 * (g0t - tan_a)
    l5 = jnp.zeros((SUBL, LANES), f32)
    for t_, r_ in ((d0, r0_ref), (d1, r1_ref), (d2, r2_ref),
                   (d3, r3_ref), (d4, r4_ref)):
        diff = jnp.abs(r_[:, :] - t_)
        l5 = l5 + jnp.where(diff < BETA, 0.5 * diff * diff / BETA,
                            diff - 0.5 * BETA)

    l5_ref[:, :] = jnp.where(valid, l5, 0.0)
    posb_ref[:, :] = jnp.where(valid & (rmax >= MD_THRES), 1.0, 0.0)


def _loss_kernel(l5_ref, posb_ref, colmax_ref, colarg_ref, out_ref):
    R = ROWS_PER_BATCH
    rowi = jax.lax.broadcasted_iota(jnp.int32, (R, LANES), 0)
    lane = jax.lax.broadcasted_iota(jnp.int32, (R, LANES), 1)
    idx = rowi * LANES + lane
    total = jnp.zeros((1, 1), jnp.float32)
    for j in range(2):
        l5 = l5_ref[pl.ds(j * R, R), :]
        pos = posb_ref[pl.ds(j * R, R), :]
        for m in range(32):
            am = colarg_ref[j, m]
            fm = jnp.where(colmax_ref[j, m] < MD_THRES, 1.0, 0.0)
            pos = jnp.maximum(pos, jnp.where(idx == am, fm, 0.0))
        S = _rsum2(pos * l5)
        num = jnp.maximum(_rsum2(pos), 1.0)
        total = total + S / (num * 5.0)
    out_ref[:, :] = total * 0.5


def kernel(regressions, anchors, refined_achors, annotations):
    f32 = jnp.float32
    B, N, _ = anchors.shape

    def acomp(x, c, pad):
        v = x[:, :, c]
        v = jnp.pad(v, ((0, 0), (0, NPAD - N)), constant_values=pad)
        return v.reshape(B * ROWS_PER_BATCH, LANES)

    a_in = [acomp(anchors, c, 1.0) for c in range(5)]
    r_in = [acomp(regressions, c, 0.0) for c in range(5)]

    gcx = annotations[:, :, 0]
    gcy = annotations[:, :, 1]
    gw = annotations[:, :, 2]
    gh = annotations[:, :, 3]
    gt = annotations[:, :, 4]
    s_g = jnp.maximum(gw, gh)
    sgx0 = gcx - s_g / 2
    sgy0 = gcy - s_g / 2
    sgx1 = gcx + s_g / 2
    sgy1 = gcy + s_g / 2
    gasq = (sgx1 - sgx0) * (sgy1 - sgy0)
    ga = gt * (jnp.pi / 180.0)
    cg, sg = jnp.cos(ga), jnp.sin(ga)
    gtan = jnp.tan(ga)
    cbx = []
    cby = []
    for dx, dy in ((-0.5, -0.5), (0.5, -0.5), (0.5, 0.5), (-0.5, 0.5)):
        cbx.append(gcx + (dx * gw) * cg - (dy * gh) * sg)
        cby.append(gcy + (dx * gw) * sg + (dy * gh) * cg)
    gab = gw * gh

    smem_arrays = [gcx, gcy, gw, gh, gtan, sgx0, sgy0, sgx1, sgy1, gasq,
                   cbx[0], cbx[1], cbx[2], cbx[3],
                   cby[0], cby[1], cby[2], cby[3], gab]

    vspec = pl.BlockSpec((SUBL, LANES), lambda g: (g, 0))
    sspec = pl.BlockSpec(memory_space=pltpu.SMEM)
    fullspec = pl.BlockSpec((2, 32), lambda g: (0, 0))

    l5, posb, colmax, colarg = pl.pallas_call(
        _pairs_kernel,
        grid=(B * NB,),
        in_specs=[vspec] * 10 + [sspec] * 19,
        out_specs=[
            vspec, vspec, fullspec, fullspec,
        ],
        out_shape=[
            jax.ShapeDtypeStruct((B * ROWS_PER_BATCH, LANES), f32),
            jax.ShapeDtypeStruct((B * ROWS_PER_BATCH, LANES), f32),
            jax.ShapeDtypeStruct((2, 32), f32),
            jax.ShapeDtypeStruct((2, 32), jnp.int32),
        ],
    )(*a_in, *r_in, *smem_arrays)

    loss = pl.pallas_call(
        _loss_kernel,
        in_specs=[
            pl.BlockSpec(memory_space=pltpu.VMEM),
            pl.BlockSpec(memory_space=pltpu.VMEM),
            sspec, sspec,
        ],
        out_specs=pl.BlockSpec(memory_space=pltpu.VMEM),
        out_shape=jax.ShapeDtypeStruct((1, 1), f32),
    )(l5, posb, colmax, colarg)
    return loss.reshape(1)


# speedup vs baseline: 38.0773x; 1.0260x over previous
import jax
import jax.numpy as jnp
from jax.experimental import pallas as pl
from jax.experimental.pallas import tpu as pltpu

MD_THRES = 0.5
BETA = 1.0 / 9.0
PI180 = 3.14159265358979323846 / 180.0

LANES = 128
SUBL = 8
TILE = LANES * SUBL
NB = 5
NPAD = TILE * NB
ROWS_PER_BATCH = NPAD // LANES


def _batcher_net(n):
    pairs = []

    def merge(lo, cnt, r):
        step = r * 2
        if step < cnt:
            merge(lo, cnt, step)
            merge(lo + r, cnt, step)
            for i in range(lo + r, lo + cnt - r, step):
                pairs.append((i, i + r))
        else:
            pairs.append((lo, lo + r))

    def sort(lo, cnt):
        if cnt > 1:
            m = cnt // 2
            sort(lo, m)
            sort(lo + m, m)
            merge(lo, cnt, 1)

    p2 = 1 << (n - 1).bit_length()
    sort(0, p2)
    return [(a, b) for (a, b) in pairs if a < n and b < n]


NET24 = _batcher_net(24)


def _rmax2(x):
    return jnp.max(jnp.max(x, axis=1, keepdims=True), axis=0, keepdims=True)


def _rmin2(x):
    return jnp.min(jnp.min(x, axis=1, keepdims=True), axis=0, keepdims=True)


def _rsum2(x):
    return jnp.sum(jnp.sum(x, axis=1, keepdims=True), axis=0, keepdims=True)


def _pairs_kernel(acx_ref, acy_ref, aw_ref, ah_ref, at_ref,
                  r0_ref, r1_ref, r2_ref, r3_ref, r4_ref,
                  gcx_ref, gcy_ref, gw_ref, gh_ref, gtan_ref,
                  sgx0_ref, sgy0_ref, sgx1_ref, sgy1_ref, gasq_ref,
                  cbx0_ref, cbx1_ref, cbx2_ref, cbx3_ref,
                  cby0_ref, cby1_ref, cby2_ref, cby3_ref,
                  gab_ref,
                  l5_ref, posb_ref, colmax_ref, colarg_ref):
    g = pl.program_id(0)
    j = g // NB
    lb = g % NB

    f32 = jnp.float32
    acx = acx_ref[:, :]
    acy = acy_ref[:, :]
    aw = aw_ref[:, :]
    ah = ah_ref[:, :]
    at = at_ref[:, :]

    rowi = jax.lax.broadcasted_iota(jnp.int32, (SUBL, LANES), 0)
    lane = jax.lax.broadcasted_iota(jnp.int32, (SUBL, LANES), 1)
    lidx = lb * TILE + rowi * LANES + lane
    valid = lidx < 5000

    s_a = jnp.maximum(aw, ah)
    ax0 = acx - s_a / 2
    ay0 = acy - s_a / 2
    ax1 = acx + s_a / 2
    ay1 = acy + s_a / 2
    area_asq = (ax1 - ax0) * (ay1 - ay0)
    area_a = aw * ah

    aa = at * PI180
    ca = jnp.cos(aa)
    sa = jnp.sin(aa)
    tan_a = jnp.tan(aa)
    hw = aw / 2
    hh = ah / 2
    DXS = (-1.0, 1.0, 1.0, -1.0)
    DYS = (-1.0, -1.0, 1.0, 1.0)
    AX = [acx + (dx * hw) * ca - (dy * hh) * sa for dx, dy in zip(DXS, DYS)]
    AY = [acy + (dx * hw) * sa + (dy * hh) * ca for dx, dy in zip(DXS, DYS)]
    EAX = [AX[(k + 1) % 4] - AX[k] for k in range(4)]
    EAY = [AY[(k + 1) % 4] - AY[k] for k in range(4)]

    @pl.when(g == 0)
    def _init():
        colmax_ref[:, :] = jnp.full((2, 32), -1.0, f32)
        colarg_ref[:, :] = jnp.zeros((2, 32), jnp.int32)

    cmv0 = colmax_ref[pl.ds(j, 1), :]
    cav0 = colarg_ref[pl.ds(j, 1), :]

    iota32 = jax.lax.broadcasted_iota(jnp.int32, (1, 32), 1)

    def compute_md(m):
        ltx = jnp.maximum(ax0, sgx0_ref[j, m])
        lty = jnp.maximum(ay0, sgy0_ref[j, m])
        rbx = jnp.minimum(ax1, sgx1_ref[j, m])
        rby = jnp.minimum(ay1, sgy1_ref[j, m])
        iw = jnp.clip(rbx - ltx, 0.0, None)
        ih = jnp.clip(rby - lty, 0.0, None)
        inter_sq = iw * ih
        union_sq = area_asq + gasq_ref[j, m] - inter_sq
        bf = inter_sq / jnp.maximum(union_sq, 1e-9)

        BX = (cbx0_ref[j, m], cbx1_ref[j, m], cbx2_ref[j, m], cbx3_ref[j, m])
        BY = (cby0_ref[j, m], cby1_ref[j, m], cby2_ref[j, m], cby3_ref[j, m])
        EBX = [BX[(k + 1) % 4] - BX[k] for k in range(4)]
        EBY = [BY[(k + 1) % 4] - BY[k] for k in range(4)]

        DX = [[BX[q] - AX[p] for q in range(4)] for p in range(4)]
        DY = [[BY[q] - AY[p] for q in range(4)] for p in range(4)]

        pts_x, pts_y, mf = [], [], []
        for p in range(4):
            ok = None
            for k in range(4):
                cr = EBY[k] * DX[p][k] - EBX[k] * DY[p][k]
                c = cr >= -1e-9
                ok = c if ok is None else (ok & c)
            pts_x.append(AX[p])
            pts_y.append(AY[p])
            mf.append(ok)
        for q in range(4):
            ok = None
            for k in range(4):
                cr = EAX[k] * DY[k][q] - EAY[k] * DX[k][q]
                c = cr >= -1e-9
                ok = c if ok is None else (ok & c)
            pts_x.append(jnp.full((SUBL, LANES), BX[q], f32))
            pts_y.append(jnp.full((SUBL, LANES), BY[q], f32))
            mf.append(ok)
        for p in range(4):
            for q in range(4):
                rx, ry = EAX[p], EAY[p]
                sx, sy = EBX[q], EBY[q]
                qpx = DX[p][q]
                qpy = DY[p][q]
                denom = rx * sy - ry * sx
                okd = jnp.abs(denom) > 1e-12
                den = jnp.where(okd, denom, 1.0)
                t = (qpx * sy - qpy * sx) / den
                u = (qpx * ry - qpy * rx) / den
                vv = okd & (t >= 0.0) & (t <= 1.0) & (u >= 0.0) & (u <= 1.0)
                pts_x.append(AX[p] + t * rx)
                pts_y.append(AY[p] + t * ry)
                mf.append(vv)

        mflt = [jnp.where(mm, 1.0, 0.0) for mm in mf]
        cnt = mflt[0]
        for k in range(1, 24):
            cnt = cnt + mflt[k]
        cntc = jnp.maximum(cnt, 1.0)
        ctrx = pts_x[0] * mflt[0]
        ctry = pts_y[0] * mflt[0]
        for k in range(1, 24):
            ctrx = ctrx + pts_x[k] * mflt[k]
            ctry = ctry + pts_y[k] * mflt[k]
        ctrx = ctrx / cntc
        ctry = ctry / cntc

        K = []
        for k in range(24):
            dx = pts_x[k] - ctrx
            dy = pts_y[k] - ctry
            sden = jnp.abs(dx) + jnp.abs(dy)
            r = dx / jnp.maximum(sden, 1e-37)
            key = jnp.where(dy >= 0.0, 1.0 - r, r - 1.0)
            K.append(jnp.where(mf[k], key, 1e9))
        X = list(pts_x)
        Y = list(pts_y)
        for a, b in NET24:
            sw = K[a] > K[b]
            ka = jnp.where(sw, K[b], K[a])
            kb = jnp.where(sw, K[a], K[b])
            xa = jnp.where(sw, X[b], X[a])
            xb = jnp.where(sw, X[a], X[b])
            ya = jnp.where(sw, Y[b], Y[a])
            yb = jnp.where(sw, Y[a], Y[b])
            K[a], K[b], X[a], X[b], Y[a], Y[b] = ka, kb, xa, xb, ya, yb
        PX = [jnp.where(cnt > k, X[k], X[0]) for k in range(24)]
        PY = [jnp.where(cnt > k, Y[k], Y[0]) for k in range(24)]
        crs = PX[23] * PY[0] - PY[23] * PX[0]
        for k in range(23):
            crs = crs + (PX[k] * PY[k + 1] - PY[k] * PX[k + 1])
        area = 0.5 * jnp.abs(crs)
        inter = jnp.where(cnt >= 3.0, area, 0.0)
        iou = inter / jnp.maximum(area_a + gab_ref[j, m] - inter, 1e-9)
        return jnp.where(bf > 0.1, iou, 0.0)

    def gt_body(i, carry):
        rmax, rarg, cmv, cav = carry
        mds = [compute_md(16 * i + t) for t in range(16)]
        for t in range(16):
            m = 16 * i + t
            md = mds[t]
            upd = md > rmax
            rmax = jnp.where(upd, md, rmax)
            rarg = jnp.where(upd, m, rarg)
            mdc = jnp.where(valid, md, -1.0)
            mx = _rmax2(mdc)
            mxb8 = jnp.broadcast_to(mx, (SUBL, LANES))
            cand = _rmin2(jnp.where(mdc == mxb8, lidx, jnp.int32(2 ** 30)))
            mxb = jnp.broadcast_to(mx, (1, 32))
            cnb = jnp.broadcast_to(cand, (1, 32))
            better = (iota32 == m) & (mxb > cmv)
            cmv = jnp.where(better, mxb, cmv)
            cav = jnp.where(better, cnb, cav)
        return rmax, rarg, cmv, cav

    rmax0 = jnp.full((SUBL, LANES), -1.0, f32)
    rarg0 = jnp.zeros((SUBL, LANES), jnp.int32)
    carry = (rmax0, rarg0, cmv0, cav0)
    for i in range(2):
        carry = gt_body(i, carry)
    rmax, rarg, cmv, cav = carry

    colmax_ref[pl.ds(j, 1), :] = cmv
    colarg_ref[pl.ds(j, 1), :] = cav

    g0x = jnp.full((SUBL, LANES), gcx_ref[j, 0], f32)
    g0y = jnp.full((SUBL, LANES), gcy_ref[j, 0], f32)
    g0w = jnp.full((SUBL, LANES), gw_ref[j, 0], f32)
    g0h = jnp.full((SUBL, LANES), gh_ref[j, 0], f32)
    g0t = jnp.full((SUBL, LANES), gtan_ref[j, 0], f32)
    for m in range(1, 32):
        sel = rarg == m
        g0x = jnp.where(sel, gcx_ref[j, m], g0x)
        g0y = jnp.where(sel, gcy_ref[j, m], g0y)
        g0w = jnp.where(sel, gw_ref[j, m], g0w)
        g0h = jnp.where(sel, gh_ref[j, m], g0h)
        g0t = jnp.where(sel, gtan_ref[j, m], g0t)

    d0 = 10.0 * (g0x - acx) / aw
    d1 = 10.0 * (g0y - acy) / ah
    d2 = 5.0 * jnp.log(g0w / aw)
    d3 = 5.0 * jnp.log(g0h / ah)
    d4 = 15.0 * (g0t - tan_a)
    l5 = jnp.zeros((SUBL, LANES), f32)
    for t_, r_ in ((d0, r0_ref), (d1, r1_ref), (d2, r2_ref),
                   (d3, r3_ref), (d4, r4_ref)):
        diff = jnp.abs(r_[:, :] - t_)
        l5 = l5 + jnp.where(diff < BETA, 0.5 * diff * diff / BETA,
                            diff - 0.5 * BETA)

    l5_ref[:, :] = jnp.where(valid, l5, 0.0)
    posb_ref[:, :] = jnp.where(valid & (rmax >= MD_THRES), 1.0, 0.0)


def _loss_kernel(l5_ref, posb_ref, colmax_ref, colarg_ref, out_ref):
    R = ROWS_PER_BATCH
    rowi = jax.lax.broadcasted_iota(jnp.int32, (R, LANES), 0)
    lane = jax.lax.broadcasted_iota(jnp.int32, (R, LANES), 1)
    idx = rowi * LANES + lane
    total = jnp.zeros((1, 1), jnp.float32)
    for j in range(2):
        l5 = l5_ref[pl.ds(j * R, R), :]
        pos = posb_ref[pl.ds(j * R, R), :]
        for m in range(32):
            am = colarg_ref[j, m]
            fm = jnp.where(colmax_ref[j, m] < MD_THRES, 1.0, 0.0)
            pos = jnp.maximum(pos, jnp.where(idx == am, fm, 0.0))
        S = _rsum2(pos * l5)
        num = jnp.maximum(_rsum2(pos), 1.0)
        total = total + S / (num * 5.0)
    out_ref[:, :] = total * 0.5


def kernel(regressions, anchors, refined_achors, annotations):
    f32 = jnp.float32
    B, N, _ = anchors.shape

    def acomp(x, c, pad):
        v = x[:, :, c]
        v = jnp.pad(v, ((0, 0), (0, NPAD - N)), constant_values=pad)
        return v.reshape(B * ROWS_PER_BATCH, LANES)

    a_in = [acomp(anchors, c, 1.0) for c in range(5)]
    r_in = [acomp(regressions, c, 0.0) for c in range(5)]

    gcx = annotations[:, :, 0]
    gcy = annotations[:, :, 1]
    gw = annotations[:, :, 2]
    gh = annotations[:, :, 3]
    gt = annotations[:, :, 4]
    s_g = jnp.maximum(gw, gh)
    sgx0 = gcx - s_g / 2
    sgy0 = gcy - s_g / 2
    sgx1 = gcx + s_g / 2
    sgy1 = gcy + s_g / 2
    gasq = (sgx1 - sgx0) * (sgy1 - sgy0)
    ga = gt * (jnp.pi / 180.0)
    cg, sg = jnp.cos(ga), jnp.sin(ga)
    gtan = jnp.tan(ga)
    cbx = []
    cby = []
    for dx, dy in ((-0.5, -0.5), (0.5, -0.5), (0.5, 0.5), (-0.5, 0.5)):
        cbx.append(gcx + (dx * gw) * cg - (dy * gh) * sg)
        cby.append(gcy + (dx * gw) * sg + (dy * gh) * cg)
    gab = gw * gh

    smem_arrays = [gcx, gcy, gw, gh, gtan, sgx0, sgy0, sgx1, sgy1, gasq,
                   cbx[0], cbx[1], cbx[2], cbx[3],
                   cby[0], cby[1], cby[2], cby[3], gab]

    vspec = pl.BlockSpec((SUBL, LANES), lambda g: (g, 0))
    sspec = pl.BlockSpec(memory_space=pltpu.SMEM)
    fullspec = pl.BlockSpec((2, 32), lambda g: (0, 0))

    l5, posb, colmax, colarg = pl.pallas_call(
        _pairs_kernel,
        grid=(B * NB,),
        in_specs=[vspec] * 10 + [sspec] * 19,
        out_specs=[
            vspec, vspec, fullspec, fullspec,
        ],
        out_shape=[
            jax.ShapeDtypeStruct((B * ROWS_PER_BATCH, LANES), f32),
            jax.ShapeDtypeStruct((B * ROWS_PER_BATCH, LANES), f32),
            jax.ShapeDtypeStruct((2, 32), f32),
            jax.ShapeDtypeStruct((2, 32), jnp.int32),
        ],
    )(*a_in, *r_in, *smem_arrays)

    loss = pl.pallas_call(
        _loss_kernel,
        in_specs=[
            pl.BlockSpec(memory_space=pltpu.VMEM),
            pl.BlockSpec(memory_space=pltpu.VMEM),
            sspec, sspec,
        ],
        out_specs=pl.BlockSpec(memory_space=pltpu.VMEM),
        out_shape=jax.ShapeDtypeStruct((1, 1), f32),
    )(l5, posb, colmax, colarg)
    return loss.reshape(1)


# speedup vs baseline: 39.0079x; 1.0244x over previous
import jax
import jax.numpy as jnp
from jax.experimental import pallas as pl
from jax.experimental.pallas import tpu as pltpu

MD_THRES = 0.5
BETA = 1.0 / 9.0
PI180 = 3.14159265358979323846 / 180.0

LANES = 128
SUBL = 8
TILE = LANES * SUBL
NB = 5
NPAD = TILE * NB
ROWS_PER_BATCH = NPAD // LANES


def _batcher_net(n):
    pairs = []

    def merge(lo, cnt, r):
        step = r * 2
        if step < cnt:
            merge(lo, cnt, step)
            merge(lo + r, cnt, step)
            for i in range(lo + r, lo + cnt - r, step):
                pairs.append((i, i + r))
        else:
            pairs.append((lo, lo + r))

    def sort(lo, cnt):
        if cnt > 1:
            m = cnt // 2
            sort(lo, m)
            sort(lo + m, m)
            merge(lo, cnt, 1)

    p2 = 1 << (n - 1).bit_length()
    sort(0, p2)
    return [(a, b) for (a, b) in pairs if a < n and b < n]


NET24 = _batcher_net(24)


def _rmax2(x):
    return jnp.max(jnp.max(x, axis=1, keepdims=True), axis=0, keepdims=True)


def _rmin2(x):
    return jnp.min(jnp.min(x, axis=1, keepdims=True), axis=0, keepdims=True)


def _rsum2(x):
    return jnp.sum(jnp.sum(x, axis=1, keepdims=True), axis=0, keepdims=True)


def _pairs_kernel(acx_ref, acy_ref, aw_ref, ah_ref, at_ref,
                  r0_ref, r1_ref, r2_ref, r3_ref, r4_ref,
                  gcx_ref, gcy_ref, gw_ref, gh_ref, gtan_ref,
                  sgx0_ref, sgy0_ref, sgx1_ref, sgy1_ref, gasq_ref,
                  cbx0_ref, cbx1_ref, cbx2_ref, cbx3_ref,
                  cby0_ref, cby1_ref, cby2_ref, cby3_ref,
                  gab_ref,
                  out_ref, l5_s, posb_s, colmax_ref, colarg_ref):
    g = pl.program_id(0)
    j = g // NB
    lb = g % NB

    f32 = jnp.float32
    acx = acx_ref[:, :]
    acy = acy_ref[:, :]
    aw = aw_ref[:, :]
    ah = ah_ref[:, :]
    at = at_ref[:, :]

    rowi = jax.lax.broadcasted_iota(jnp.int32, (SUBL, LANES), 0)
    lane = jax.lax.broadcasted_iota(jnp.int32, (SUBL, LANES), 1)
    lidx = lb * TILE + rowi * LANES + lane
    valid = lidx < 5000

    s_a = jnp.maximum(aw, ah)
    ax0 = acx - s_a / 2
    ay0 = acy - s_a / 2
    ax1 = acx + s_a / 2
    ay1 = acy + s_a / 2
    area_asq = (ax1 - ax0) * (ay1 - ay0)
    area_a = aw * ah

    aa = at * PI180
    ca = jnp.cos(aa)
    sa = jnp.sin(aa)
    tan_a = jnp.tan(aa)
    hw = aw / 2
    hh = ah / 2
    DXS = (-1.0, 1.0, 1.0, -1.0)
    DYS = (-1.0, -1.0, 1.0, 1.0)
    AX = [acx + (dx * hw) * ca - (dy * hh) * sa for dx, dy in zip(DXS, DYS)]
    AY = [acy + (dx * hw) * sa + (dy * hh) * ca for dx, dy in zip(DXS, DYS)]
    EAX = [AX[(k + 1) % 4] - AX[k] for k in range(4)]
    EAY = [AY[(k + 1) % 4] - AY[k] for k in range(4)]

    @pl.when(g == 0)
    def _init():
        colmax_ref[:, :] = jnp.full((2, 32), -1.0, f32)
        colarg_ref[:, :] = jnp.zeros((2, 32), jnp.int32)

    cmv0 = colmax_ref[pl.ds(j, 1), :]
    cav0 = colarg_ref[pl.ds(j, 1), :]

    iota32 = jax.lax.broadcasted_iota(jnp.int32, (1, 32), 1)

    def compute_md(m):
        ltx = jnp.maximum(ax0, sgx0_ref[j, m])
        lty = jnp.maximum(ay0, sgy0_ref[j, m])
        rbx = jnp.minimum(ax1, sgx1_ref[j, m])
        rby = jnp.minimum(ay1, sgy1_ref[j, m])
        iw = jnp.clip(rbx - ltx, 0.0, None)
        ih = jnp.clip(rby - lty, 0.0, None)
        inter_sq = iw * ih
        union_sq = area_asq + gasq_ref[j, m] - inter_sq
        bf = inter_sq / jnp.maximum(union_sq, 1e-9)

        BX = (cbx0_ref[j, m], cbx1_ref[j, m], cbx2_ref[j, m], cbx3_ref[j, m])
        BY = (cby0_ref[j, m], cby1_ref[j, m], cby2_ref[j, m], cby3_ref[j, m])
        EBX = [BX[(k + 1) % 4] - BX[k] for k in range(4)]
        EBY = [BY[(k + 1) % 4] - BY[k] for k in range(4)]

        DX = [[BX[q] - AX[p] for q in range(4)] for p in range(4)]
        DY = [[BY[q] - AY[p] for q in range(4)] for p in range(4)]

        pts_x, pts_y, mf = [], [], []
        for p in range(4):
            ok = None
            for k in range(4):
                cr = EBY[k] * DX[p][k] - EBX[k] * DY[p][k]
                c = cr >= -1e-9
                ok = c if ok is None else (ok & c)
            pts_x.append(AX[p])
            pts_y.append(AY[p])
            mf.append(ok)
        for q in range(4):
            ok = None
            for k in range(4):
                cr = EAX[k] * DY[k][q] - EAY[k] * DX[k][q]
                c = cr >= -1e-9
                ok = c if ok is None else (ok & c)
            pts_x.append(jnp.full((SUBL, LANES), BX[q], f32))
            pts_y.append(jnp.full((SUBL, LANES), BY[q], f32))
            mf.append(ok)
        for p in range(4):
            for q in range(4):
                rx, ry = EAX[p], EAY[p]
                sx, sy = EBX[q], EBY[q]
                qpx = DX[p][q]
                qpy = DY[p][q]
                denom = rx * sy - ry * sx
                okd = jnp.abs(denom) > 1e-12
                den = jnp.where(okd, denom, 1.0)
                t = (qpx * sy - qpy * sx) / den
                u = (qpx * ry - qpy * rx) / den
                vv = okd & (t >= 0.0) & (t <= 1.0) & (u >= 0.0) & (u <= 1.0)
                pts_x.append(AX[p] + t * rx)
                pts_y.append(AY[p] + t * ry)
                mf.append(vv)

        mflt = [jnp.where(mm, 1.0, 0.0) for mm in mf]
        cnt = mflt[0]
        for k in range(1, 24):
            cnt = cnt + mflt[k]
        cntc = jnp.maximum(cnt, 1.0)
        ctrx = pts_x[0] * mflt[0]
        ctry = pts_y[0] * mflt[0]
        for k in range(1, 24):
            ctrx = ctrx + pts_x[k] * mflt[k]
            ctry = ctry + pts_y[k] * mflt[k]
        ctrx = ctrx / cntc
        ctry = ctry / cntc

        K = []
        for k in range(24):
            dx = pts_x[k] - ctrx
            dy = pts_y[k] - ctry
            sden = jnp.abs(dx) + jnp.abs(dy)
            r = dx / jnp.maximum(sden, 1e-37)
            key = jnp.where(dy >= 0.0, 1.0 - r, r - 1.0)
            K.append(jnp.where(mf[k], key, 1e9))
        X = list(pts_x)
        Y = list(pts_y)
        for a, b in NET24:
            sw = K[a] > K[b]
            ka = jnp.where(sw, K[b], K[a])
            kb = jnp.where(sw, K[a], K[b])
            xa = jnp.where(sw, X[b], X[a])
            xb = jnp.where(sw, X[a], X[b])
            ya = jnp.where(sw, Y[b], Y[a])
            yb = jnp.where(sw, Y[a], Y[b])
            K[a], K[b], X[a], X[b], Y[a], Y[b] = ka, kb, xa, xb, ya, yb
        PX = [jnp.where(cnt > k, X[k], X[0]) for k in range(24)]
        PY = [jnp.where(cnt > k, Y[k], Y[0]) for k in range(24)]
        crs = PX[23] * PY[0] - PY[23] * PX[0]
        for k in range(23):
            crs = crs + (PX[k] * PY[k + 1] - PY[k] * PX[k + 1])
        area = 0.5 * jnp.abs(crs)
        inter = jnp.where(cnt >= 3.0, area, 0.0)
        iou = inter / jnp.maximum(area_a + gab_ref[j, m] - inter, 1e-9)
        return jnp.where(bf > 0.1, iou, 0.0)

    def gt_body(i, carry):
        rmax, rarg, cmv, cav = carry
        mds = [compute_md(16 * i + t) for t in range(16)]
        for t in range(16):
            m = 16 * i + t
            md = mds[t]
            upd = md > rmax
            rmax = jnp.where(upd, md, rmax)
            rarg = jnp.where(upd, m, rarg)
            mdc = jnp.where(valid, md, -1.0)
            mx = _rmax2(mdc)
            mxb8 = jnp.broadcast_to(mx, (SUBL, LANES))
            cand = _rmin2(jnp.where(mdc == mxb8, lidx, jnp.int32(2 ** 30)))
            mxb = jnp.broadcast_to(mx, (1, 32))
            cnb = jnp.broadcast_to(cand, (1, 32))
            better = (iota32 == m) & (mxb > cmv)
            cmv = jnp.where(better, mxb, cmv)
            cav = jnp.where(better, cnb, cav)
        return rmax, rarg, cmv, cav

    rmax0 = jnp.full((SUBL, LANES), -1.0, f32)
    rarg0 = jnp.zeros((SUBL, LANES), jnp.int32)
    carry = (rmax0, rarg0, cmv0, cav0)
    for i in range(2):
        carry = gt_body(i, carry)
    rmax, rarg, cmv, cav = carry

    colmax_ref[pl.ds(j, 1), :] = cmv
    colarg_ref[pl.ds(j, 1), :] = cav

    g0x = jnp.full((SUBL, LANES), gcx_ref[j, 0], f32)
    g0y = jnp.full((SUBL, LANES), gcy_ref[j, 0], f32)
    g0w = jnp.full((SUBL, LANES), gw_ref[j, 0], f32)
    g0h = jnp.full((SUBL, LANES), gh_ref[j, 0], f32)
    g0t = jnp.full((SUBL, LANES), gtan_ref[j, 0], f32)
    for m in range(1, 32):
        sel = rarg == m
        g0x = jnp.where(sel, gcx_ref[j, m], g0x)
        g0y = jnp.where(sel, gcy_ref[j, m], g0y)
        g0w = jnp.where(sel, gw_ref[j, m], g0w)
        g0h = jnp.where(sel, gh_ref[j, m], g0h)
        g0t = jnp.where(sel, gtan_ref[j, m], g0t)

    d0 = 10.0 * (g0x - acx) / aw
    d1 = 10.0 * (g0y - acy) / ah
    d2 = 5.0 * jnp.log(g0w / aw)
    d3 = 5.0 * jnp.log(g0h / ah)
    d4 = 15.0 * (g0t - tan_a)
    l5 = jnp.zeros((SUBL, LANES), f32)
    for t_, r_ in ((d0, r0_ref), (d1, r1_ref), (d2, r2_ref),
                   (d3, r3_ref), (d4, r4_ref)):
        diff = jnp.abs(r_[:, :] - t_)
        l5 = l5 + jnp.where(diff < BETA, 0.5 * diff * diff / BETA,
                            diff - 0.5 * BETA)

    l5_s[pl.ds(g * SUBL, SUBL), :] = jnp.where(valid, l5, 0.0)
    posb_s[pl.ds(g * SUBL, SUBL), :] = jnp.where(
        valid & (rmax >= MD_THRES), 1.0, 0.0)

    @pl.when(g == 2 * NB - 1)
    def _finalize():
        R = ROWS_PER_BATCH
        ri = jax.lax.broadcasted_iota(jnp.int32, (R, LANES), 0)
        la = jax.lax.broadcasted_iota(jnp.int32, (R, LANES), 1)
        idx = ri * LANES + la
        total = jnp.zeros((1, 1), jnp.float32)
        for jj in range(2):
            l5j = l5_s[pl.ds(jj * R, R), :]
            pos = posb_s[pl.ds(jj * R, R), :]
            cmrow = colmax_ref[pl.ds(jj, 1), :]
            carow = colarg_ref[pl.ds(jj, 1), :]
            force = jnp.where(cmrow < MD_THRES, 1.0, 0.0)
            for m in range(32):
                am = jax.lax.slice(carow, (0, m), (1, m + 1))
                fm = jax.lax.slice(force, (0, m), (1, m + 1))
                amb = jnp.broadcast_to(am, (R, LANES))
                fmb = jnp.broadcast_to(fm, (R, LANES))
                pos = jnp.maximum(pos, jnp.where(idx == amb, fmb, 0.0))
            S = _rsum2(pos * l5j)
            num = jnp.maximum(_rsum2(pos), 1.0)
            total = total + S / (num * 5.0)
        out_ref[:, :] = total * 0.5


def kernel(regressions, anchors, refined_achors, annotations):
    f32 = jnp.float32
    B, N, _ = anchors.shape

    def acomp(x, c, pad):
        v = x[:, :, c]
        v = jnp.pad(v, ((0, 0), (0, NPAD - N)), constant_values=pad)
        return v.reshape(B * ROWS_PER_BATCH, LANES)

    a_in = [acomp(anchors, c, 1.0) for c in range(5)]
    r_in = [acomp(regressions, c, 0.0) for c in range(5)]

    gcx = annotations[:, :, 0]
    gcy = annotations[:, :, 1]
    gw = annotations[:, :, 2]
    gh = annotations[:, :, 3]
    gt = annotations[:, :, 4]
    s_g = jnp.maximum(gw, gh)
    sgx0 = gcx - s_g / 2
    sgy0 = gcy - s_g / 2
    sgx1 = gcx + s_g / 2
    sgy1 = gcy + s_g / 2
    gasq = (sgx1 - sgx0) * (sgy1 - sgy0)
    ga = gt * (jnp.pi / 180.0)
    cg, sg = jnp.cos(ga), jnp.sin(ga)
    gtan = jnp.tan(ga)
    cbx = []
    cby = []
    for dx, dy in ((-0.5, -0.5), (0.5, -0.5), (0.5, 0.5), (-0.5, 0.5)):
        cbx.append(gcx + (dx * gw) * cg - (dy * gh) * sg)
        cby.append(gcy + (dx * gw) * sg + (dy * gh) * cg)
    gab = gw * gh

    smem_arrays = [gcx, gcy, gw, gh, gtan, sgx0, sgy0, sgx1, sgy1, gasq,
                   cbx[0], cbx[1], cbx[2], cbx[3],
                   cby[0], cby[1], cby[2], cby[3], gab]

    vspec = pl.BlockSpec((SUBL, LANES), lambda g: (g, 0))
    sspec = pl.BlockSpec(memory_space=pltpu.SMEM)

    loss = pl.pallas_call(
        _pairs_kernel,
        grid=(B * NB,),
        in_specs=[vspec] * 10 + [sspec] * 19,
        out_specs=pl.BlockSpec((1, 1), lambda g: (0, 0)),
        out_shape=jax.ShapeDtypeStruct((1, 1), f32),
        scratch_shapes=[
            pltpu.VMEM((B * ROWS_PER_BATCH, LANES), f32),
            pltpu.VMEM((B * ROWS_PER_BATCH, LANES), f32),
            pltpu.VMEM((2, 32), f32),
            pltpu.VMEM((2, 32), jnp.int32),
        ],
    )(*a_in, *r_in, *smem_arrays)
    return loss.reshape(1)
